# Initial kernel scaffold; baseline (speedup 1.0000x reference)
import jax
import jax.numpy as jnp
from jax.experimental import pallas as pl


def kernel(x_paper, x_author, edge_index, edge_type, proj_w_paper, proj_b_paper, proj_w_author, proj_b_author, comp1, basis1, root1, comp2, basis2, root2):
    raise NotImplementedError("write your pallas kernel here")


# speedup vs baseline: 10.0721x; 10.0721x over previous
import functools

import jax
import jax.numpy as jnp
from jax import lax
from jax.experimental import pallas as pl
from jax.experimental.pallas import tpu as pltpu
from jax.experimental.pallas import tpu_sc as plsc

N = 10000
E = 320000
R = 4
NB = 8
HID = 128
EPS = 1e-5

NR = N * R
NRP = 40960
NPAD = NRP // R
NCORE = 2
NSUB = 16
NWORK = NCORE * NSUB
EPADDED = 327680
EXTRA = EPADDED - E
CHUNK = 128
NCHUNK = EPADDED // (NWORK * CHUNK)
PASSES = 4
PCOL = HID // PASSES
ROWBLK = 1000


def _prep_body(src_ref, dst_ref, et_ref, gid_ref, tid_ref):
    et = et_ref[...]
    gid_ref[...] = src_ref[...] * R + et
    tid_ref[...] = dst_ref[...] * R + et


def _prep_ids(src, dst, et):
    shp = jax.ShapeDtypeStruct((2500, HID), jnp.int32)
    return pl.pallas_call(
        _prep_body,
        out_shape=(shp, shp),
    )(src.reshape(2500, HID), dst.reshape(2500, HID), et.reshape(2500, HID))


def _wmix_body(comp_ref, basis_ref, w_ref):
    w_ref[0] = jnp.dot(comp_ref[0], basis_ref[0],
                       preferred_element_type=jnp.float32)


def _wmix(comp, basisr):
    return pl.pallas_call(
        _wmix_body,
        grid=(2,),
        in_specs=[
            pl.BlockSpec((1, R, NB), lambda g: (g, 0, 0)),
            pl.BlockSpec((1, NB, HID * HID), lambda g: (g, 0, 0)),
        ],
        out_specs=pl.BlockSpec((1, R, HID * HID), lambda g: (g, 0, 0)),
        out_shape=jax.ShapeDtypeStruct((2, R, HID * HID), jnp.float32),
    )(comp, basisr)


def _proj_body(x_ref, w_ref, b_ref, y_ref):
    y = jnp.dot(x_ref[0], w_ref[0], preferred_element_type=jnp.float32)
    y_ref[0] = jnp.maximum(y + b_ref[0, 0:1, :], 0.0)


def _proj(xs, ws, bs):
    nb = 5
    blk = 5000 // nb
    return pl.pallas_call(
        _proj_body,
        grid=(2, nb),
        in_specs=[
            pl.BlockSpec((1, blk, HID), lambda t, g: (t, g, 0)),
            pl.BlockSpec((1, HID, HID), lambda t, g: (t, 0, 0)),
            pl.BlockSpec((1, 8, HID), lambda t, g: (t, 0, 0)),
        ],
        out_specs=pl.BlockSpec((1, blk, HID), lambda t, g: (t, g, 0)),
        out_shape=jax.ShapeDtypeStruct((2, 5000, HID), jnp.float32),
    )(xs, ws, bs)


def _hmm_body(x_ref, w_ref, o0_ref, y0_ref, y1_ref, y2_ref, y3_ref):
    h = jnp.dot(x_ref[...], w_ref[...], preferred_element_type=jnp.float32)
    o0_ref[...] = h[:, 0:HID]
    y0_ref[...] = h[:, HID:2 * HID]
    y1_ref[...] = h[:, 2 * HID:3 * HID]
    y2_ref[...] = h[:, 3 * HID:4 * HID]
    y3_ref[...] = h[:, 4 * HID:5 * HID]


def _hmm(x, wfull):
    nb = N // ROWBLK
    shp = jax.ShapeDtypeStruct((N, HID), jnp.float32)
    return pl.pallas_call(
        _hmm_body,
        grid=(nb,),
        in_specs=[
            pl.BlockSpec((ROWBLK, HID), lambda g: (g, 0)),
            pl.BlockSpec((HID, 5 * HID), lambda g: (0, 0)),
        ],
        out_specs=[pl.BlockSpec((ROWBLK, HID), lambda g: (g, 0))] * 5,
        out_shape=(shp,) * 5,
    )(x, wfull)


def _combine_body(relu, pp_ref, o0_ref, deg_ref, y_ref, cs_ref, sq_ref):
    g = pl.program_id(0)
    inv = 1.0 / jnp.maximum(deg_ref[0] + deg_ref[1], 1.0)
    parts = []
    for p in range(PASSES):
        w = (pp_ref[0, p] + pp_ref[1, p]) * inv
        t = w[:, 0:PCOL]
        for r in range(1, R):
            t = t + w[:, r * PCOL:(r + 1) * PCOL]
        parts.append(t)
    y = o0_ref[...] + jnp.concatenate(parts, axis=1)
    if relu:
        y = jnp.maximum(y, 0.0)
    y_ref[...] = y
    cs = jnp.broadcast_to(jnp.sum(y, axis=0, keepdims=True), (8, HID))
    sq = jnp.full((8, HID), jnp.sum(y * y))

    @pl.when(g == 0)
    def _():
        cs_ref[...] = cs
        sq_ref[...] = sq

    @pl.when(g > 0)
    def _():
        cs_ref[...] += cs
        sq_ref[...] += sq


def _combine(pp, out0, degx, relu):
    nb = N // ROWBLK
    small = jax.ShapeDtypeStruct((8, HID), jnp.float32)
    return pl.pallas_call(
        functools.partial(_combine_body, relu),
        grid=(nb,),
        in_specs=[
            pl.BlockSpec((2, PASSES, ROWBLK, HID), lambda g: (0, 0, g, 0)),
            pl.BlockSpec((ROWBLK, HID), lambda g: (g, 0)),
            pl.BlockSpec((2, ROWBLK, HID), lambda g: (0, g, 0)),
        ],
        out_specs=[
            pl.BlockSpec((ROWBLK, HID), lambda g: (g, 0)),
            pl.BlockSpec((8, HID), lambda g: (0, 0)),
            pl.BlockSpec((8, HID), lambda g: (0, 0)),
        ],
        out_shape=(jax.ShapeDtypeStruct((N, HID), jnp.float32), small, small),
    )(pp, out0, degx)


def _scale_body(y_ref, cs_ref, sq_ref, o_ref):
    mu = cs_ref[0:1, :] * (1.0 / N)
    var = sq_ref[0, 0] * (1.0 / N) - jnp.sum(mu * mu)
    s = jax.lax.rsqrt(EPS + var)
    o_ref[...] = (y_ref[...] - mu) * s


def _scale(y, cs, sq):
    nb = N // ROWBLK
    return pl.pallas_call(
        _scale_body,
        grid=(nb,),
        in_specs=[
            pl.BlockSpec((ROWBLK, HID), lambda g: (g, 0)),
            pl.BlockSpec((8, HID), lambda g: (0, 0)),
            pl.BlockSpec((8, HID), lambda g: (0, 0)),
        ],
        out_specs=pl.BlockSpec((ROWBLK, HID), lambda g: (g, 0)),
        out_shape=jax.ShapeDtypeStruct((N, HID), jnp.float32),
    )(y, cs, sq)


_MESH = plsc.VectorSubcoreMesh(core_axis_name="c", subcore_axis_name="s",
                               num_cores=NCORE, num_subcores=NSUB)
_SLICE = NRP // NSUB


def _deg_kernel(tid_hbm, ones_hbm, deg_hbm, tidv, ones_v, deg_sh, zrow):
    c = lax.axis_index("c")
    s = lax.axis_index("s")
    wid = s * NCORE + c

    pltpu.sync_copy(tid_hbm.at[wid], tidv)
    pltpu.sync_copy(ones_hbm.at[pl.ds(0, CHUNK)], ones_v)
    @pl.loop(0, 16)
    def _(i):
        zrow[...] = jnp.zeros((16,), jnp.float32)
        del i
    @pl.loop(0, _SLICE // 16)
    def _(i):
        pltpu.sync_copy(zrow, deg_sh.at[pl.ds(s * _SLICE + i * 16, 16)])
    plsc.subcore_barrier()

    @pl.loop(0, NCHUNK)
    def _(k):
        pltpu.sync_copy(ones_v, deg_sh.at[tidv.at[k]], add=True)

    plsc.subcore_barrier()
    pltpu.sync_copy(deg_sh.at[pl.ds(s * _SLICE, _SLICE)],
                    deg_hbm.at[c, pl.ds(s * _SLICE, _SLICE)])


def _deg(tid3, ones):
    return pl.kernel(
        _deg_kernel,
        out_type=jax.ShapeDtypeStruct((NCORE, NRP), jnp.float32),
        mesh=_MESH,
        scratch_types=[
            pltpu.VMEM((NCHUNK, CHUNK), jnp.int32),
            pltpu.VMEM((CHUNK,), jnp.float32),
            pltpu.VMEM_SHARED((NRP,), jnp.float32),
            pltpu.VMEM((16,), jnp.float32),
        ],
    )(tid3, ones)


def _edge_kernel(t0, t1, t2, t3, gid_hbm, tid_hbm, zeros_hbm, pp_hbm,
                 gidv, tidv, rows, acc, sem):
    c = lax.axis_index("c")
    s = lax.axis_index("s")
    wid = s * NCORE + c
    tabs = (t0, t1, t2, t3)

    pltpu.sync_copy(gid_hbm.at[wid], gidv)
    pltpu.sync_copy(tid_hbm.at[wid], tidv)

    for p in range(PASSES):
        pltpu.sync_copy(zeros_hbm.at[pl.ds(s * _SLICE, _SLICE)],
                        acc.at[pl.ds(s * _SLICE, _SLICE)])
        plsc.subcore_barrier()

        @pl.loop(0, NCHUNK)
        def _(k):
            pltpu.async_copy(tabs[p].at[gidv.at[k]], rows, sem).wait()
            pltpu.sync_copy(rows, acc.at[tidv.at[k]], add=True)

        plsc.subcore_barrier()
        pltpu.sync_copy(acc.at[pl.ds(s * _SLICE, _SLICE)],
                        pp_hbm.at[c, p, pl.ds(s * _SLICE, _SLICE)])
        plsc.subcore_barrier()


def _edge(tabs, gid3, tid3, zeros):
    return pl.kernel(
        _edge_kernel,
        out_type=jax.ShapeDtypeStruct((NCORE, PASSES, NRP, PCOL),
                                      jnp.float32),
        mesh=_MESH,
        compiler_params=pltpu.CompilerParams(use_tc_tiling_on_sc=False),
        scratch_types=[
            pltpu.VMEM((NCHUNK, CHUNK), jnp.int32),
            pltpu.VMEM((NCHUNK, CHUNK), jnp.int32),
            pltpu.VMEM((CHUNK, PCOL), jnp.float32),
            pltpu.VMEM_SHARED((NRP, PCOL), jnp.float32),
            pltpu.SemaphoreType.DMA,
        ],
    )(tabs[0], tabs[1], tabs[2], tabs[3], gid3, tid3, zeros)


def kernel(x_paper, x_author, edge_index, edge_type,
           proj_w_paper, proj_b_paper, proj_w_author, proj_b_author,
           comp1, basis1, root1, comp2, basis2, root2):
    src = edge_index[0]
    dst = edge_index[1]

    gid, tid = _prep_ids(src, dst, edge_type)
    k = jnp.arange(EXTRA, dtype=jnp.int32)
    pad_gid = (k * 7919) % NR
    pad_tid = NR + k % (NRP - NR)
    gid3 = jnp.concatenate([gid.reshape(E), pad_gid]).reshape(
        NWORK, NCHUNK, CHUNK)
    tid3 = jnp.concatenate([tid.reshape(E), pad_tid]).reshape(
        NWORK, NCHUNK, CHUNK)

    ones = jnp.ones((CHUNK,), jnp.float32)
    zeros = jnp.zeros((NRP, PCOL), jnp.float32)

    degp = _deg(tid3, ones)
    degx = jnp.repeat(degp.reshape(NCORE, NPAD, R), PCOL, axis=2)

    xs = jnp.stack([x_paper, x_author])
    ws = jnp.stack([proj_w_paper, proj_w_author])
    bs = jnp.broadcast_to(jnp.stack([proj_b_paper, proj_b_author])[:, None, :],
                          (2, 8, HID))
    x = _proj(xs, ws, bs).reshape(N, HID)

    comps = jnp.stack([comp1, comp2])
    basisr = jnp.stack([basis1.reshape(NB, HID * HID),
                        basis2.reshape(NB, HID * HID)])
    wall = _wmix(comps, basisr)

    def wfull(l, root):
        wb = wall[l].reshape(R, HID, PASSES, PCOL).transpose(1, 2, 0, 3)
        return jnp.concatenate([root, wb.reshape(HID, R * HID)], axis=1)

    def layer(xin, root, l, relu):
        o0, y0, y1, y2, y3 = _hmm(xin, wfull(l, root))
        tabs = [y.reshape(NR, PCOL) for y in (y0, y1, y2, y3)]
        pp = _edge(tabs, gid3, tid3, zeros)
        ppv = pp.reshape(NCORE, PASSES, NPAD, HID)
        y, cs, sq = _combine(ppv, o0, degx, relu)
        return _scale(y, cs, sq)

    h = layer(x, root1, 0, relu=True)
    out = layer(h, root2, 1, relu=False)
    return (out, h)


# speedup vs baseline: 14.8363x; 1.4730x over previous
import functools

import jax
import jax.numpy as jnp
from jax import lax
from jax.experimental import pallas as pl
from jax.experimental.pallas import tpu as pltpu
from jax.experimental.pallas import tpu_sc as plsc

N = 10000
E = 320000
R = 4
NB = 8
HID = 128
EPS = 1e-5

NR = N * R
NRP = 40960
NPAD = NRP // R
NCORE = 2
NSUB = 16
NWORK = NCORE * NSUB
EPADDED = 327680
EXTRA = EPADDED - E
CHUNK = 128
NCHUNK = EPADDED // (NWORK * CHUNK)
PASSES = 4
PCOL = HID // PASSES
ROWBLK = 1000


def _prep_body(src_ref, dst_ref, et_ref, gid_ref, tid_ref):
    et = et_ref[...]
    gid_ref[...] = src_ref[...] * R + et
    tid_ref[...] = dst_ref[...] * R + et


def _prep_ids(src, dst, et):
    shp = jax.ShapeDtypeStruct((2500, HID), jnp.int32)
    return pl.pallas_call(
        _prep_body,
        out_shape=(shp, shp),
    )(src.reshape(2500, HID), dst.reshape(2500, HID), et.reshape(2500, HID))


def _wmix_body(comp_ref, basis_ref, w_ref):
    w_ref[0] = jnp.dot(comp_ref[0], basis_ref[0],
                       preferred_element_type=jnp.float32)


def _wmix(comp, basisr):
    return pl.pallas_call(
        _wmix_body,
        grid=(2,),
        in_specs=[
            pl.BlockSpec((1, R, NB), lambda g: (g, 0, 0)),
            pl.BlockSpec((1, NB, HID * HID), lambda g: (g, 0, 0)),
        ],
        out_specs=pl.BlockSpec((1, R, HID * HID), lambda g: (g, 0, 0)),
        out_shape=jax.ShapeDtypeStruct((2, R, HID * HID), jnp.float32),
    )(comp, basisr)


def _proj_body(x_ref, w_ref, b_ref, y_ref):
    y = jnp.dot(x_ref[0], w_ref[0], preferred_element_type=jnp.float32)
    y_ref[0] = jnp.maximum(y + b_ref[0, 0:1, :], 0.0)


def _proj(xs, ws, bs):
    nb = 5
    blk = 5000 // nb
    return pl.pallas_call(
        _proj_body,
        grid=(2, nb),
        in_specs=[
            pl.BlockSpec((1, blk, HID), lambda t, g: (t, g, 0)),
            pl.BlockSpec((1, HID, HID), lambda t, g: (t, 0, 0)),
            pl.BlockSpec((1, 8, HID), lambda t, g: (t, 0, 0)),
        ],
        out_specs=pl.BlockSpec((1, blk, HID), lambda t, g: (t, g, 0)),
        out_shape=jax.ShapeDtypeStruct((2, 5000, HID), jnp.float32),
    )(xs, ws, bs)


def _hmm_body(x_ref, w_ref, o0_ref, y0_ref, y1_ref, y2_ref, y3_ref):
    h = jnp.dot(x_ref[...], w_ref[...], preferred_element_type=jnp.float32)
    o0_ref[...] = h[:, 0:HID]
    y0_ref[...] = h[:, HID:2 * HID]
    y1_ref[...] = h[:, 2 * HID:3 * HID]
    y2_ref[...] = h[:, 3 * HID:4 * HID]
    y3_ref[...] = h[:, 4 * HID:5 * HID]


def _hmm(x, wfull):
    nb = N // ROWBLK
    shp = jax.ShapeDtypeStruct((N, HID), jnp.float32)
    return pl.pallas_call(
        _hmm_body,
        grid=(nb,),
        in_specs=[
            pl.BlockSpec((ROWBLK, HID), lambda g: (g, 0)),
            pl.BlockSpec((HID, 5 * HID), lambda g: (0, 0)),
        ],
        out_specs=[pl.BlockSpec((ROWBLK, HID), lambda g: (g, 0))] * 5,
        out_shape=(shp,) * 5,
    )(x, wfull)


def _combine_body(relu, pp_ref, o0_ref, deg_ref, y_ref, cs_ref, sq_ref):
    g = pl.program_id(0)
    inv = 1.0 / jnp.maximum(deg_ref[0] + deg_ref[1], 1.0)
    parts = []
    for p in range(PASSES):
        w = (pp_ref[0, p] + pp_ref[1, p]) * inv
        t = w[:, 0:PCOL]
        for r in range(1, R):
            t = t + w[:, r * PCOL:(r + 1) * PCOL]
        parts.append(t)
    y = o0_ref[...] + jnp.concatenate(parts, axis=1)
    if relu:
        y = jnp.maximum(y, 0.0)
    y_ref[...] = y
    cs = jnp.broadcast_to(jnp.sum(y, axis=0, keepdims=True), (8, HID))
    sq = jnp.full((8, HID), jnp.sum(y * y))

    @pl.when(g == 0)
    def _():
        cs_ref[...] = cs
        sq_ref[...] = sq

    @pl.when(g > 0)
    def _():
        cs_ref[...] += cs
        sq_ref[...] += sq


def _combine(pp, out0, degx, relu):
    nb = N // ROWBLK
    small = jax.ShapeDtypeStruct((8, HID), jnp.float32)
    return pl.pallas_call(
        functools.partial(_combine_body, relu),
        grid=(nb,),
        in_specs=[
            pl.BlockSpec((2, PASSES, ROWBLK, HID), lambda g: (0, 0, g, 0)),
            pl.BlockSpec((ROWBLK, HID), lambda g: (g, 0)),
            pl.BlockSpec((2, ROWBLK, HID), lambda g: (0, g, 0)),
        ],
        out_specs=[
            pl.BlockSpec((ROWBLK, HID), lambda g: (g, 0)),
            pl.BlockSpec((8, HID), lambda g: (0, 0)),
            pl.BlockSpec((8, HID), lambda g: (0, 0)),
        ],
        out_shape=(jax.ShapeDtypeStruct((N, HID), jnp.float32), small, small),
    )(pp, out0, degx)


def _scale_body(y_ref, cs_ref, sq_ref, o_ref):
    mu = cs_ref[0:1, :] * (1.0 / N)
    var = sq_ref[0, 0] * (1.0 / N) - jnp.sum(mu * mu)
    s = jax.lax.rsqrt(EPS + var)
    o_ref[...] = (y_ref[...] - mu) * s


def _scale(y, cs, sq):
    nb = N // ROWBLK
    return pl.pallas_call(
        _scale_body,
        grid=(nb,),
        in_specs=[
            pl.BlockSpec((ROWBLK, HID), lambda g: (g, 0)),
            pl.BlockSpec((8, HID), lambda g: (0, 0)),
            pl.BlockSpec((8, HID), lambda g: (0, 0)),
        ],
        out_specs=pl.BlockSpec((ROWBLK, HID), lambda g: (g, 0)),
        out_shape=jax.ShapeDtypeStruct((N, HID), jnp.float32),
    )(y, cs, sq)


_MESH = plsc.VectorSubcoreMesh(core_axis_name="c", subcore_axis_name="s",
                               num_cores=NCORE, num_subcores=NSUB)
_SLICE = NRP // NSUB


_ZROWS = 320


def _edge_kernel(with_deg, t0, t1, t2, t3, gid_hbm, tid_hbm, pp_hbm, deg_hbm,
                 gidv, tidv, rows_a, rows_b, zbuf, zbuf1, ones_v, acc, deg_sh,
                 sem_a, sem_b):
    c = lax.axis_index("c")
    s = lax.axis_index("s")
    wid = s * NCORE + c
    tabs = (t0, t1, t2, t3)

    pltpu.sync_copy(gid_hbm.at[wid], gidv)
    pltpu.sync_copy(tid_hbm.at[wid], tidv)

    @pl.loop(0, _ZROWS)
    def _(i):
        zbuf[i, pl.ds(0, 16)] = jnp.zeros((16,), jnp.float32)
        zbuf[i, pl.ds(16, 16)] = jnp.zeros((16,), jnp.float32)
    if with_deg:
        @pl.loop(0, CHUNK // 16)
        def _(i):
            ones_v[pl.ds(i * 16, 16)] = jnp.ones((16,), jnp.float32)
        @pl.loop(0, _SLICE // 16)
        def _(i):
            zbuf1[pl.ds(i * 16, 16)] = jnp.zeros((16,), jnp.float32)
        pltpu.sync_copy(zbuf1, deg_sh.at[pl.ds(s * _SLICE, _SLICE)])
    del wid

    for p in range(PASSES):
        @pl.loop(0, _SLICE // _ZROWS)
        def _(j):
            pltpu.sync_copy(
                zbuf, acc.at[pl.ds(s * _SLICE + j * _ZROWS, _ZROWS)])
        plsc.subcore_barrier()

        tab = tabs[p]

        pltpu.async_copy(tab.at[gidv.at[0]], rows_a, sem_a)
        pltpu.async_copy(tab.at[gidv.at[1]], rows_b, sem_b)

        @pl.loop(0, NCHUNK, step=2)
        def _(k):
            pltpu.make_async_copy(tab.at[gidv.at[k]], rows_a, sem_a).wait()
            pltpu.sync_copy(rows_a, acc.at[tidv.at[k]], add=True)
            if with_deg and p == 0:
                pltpu.sync_copy(ones_v, deg_sh.at[tidv.at[k]], add=True)

            @pl.when(k + 2 < NCHUNK)
            def _():
                pltpu.async_copy(tab.at[gidv.at[k + 2]], rows_a, sem_a)

            pltpu.make_async_copy(tab.at[gidv.at[k + 1]], rows_b,
                                  sem_b).wait()
            pltpu.sync_copy(rows_b, acc.at[tidv.at[k + 1]], add=True)
            if with_deg and p == 0:
                pltpu.sync_copy(ones_v, deg_sh.at[tidv.at[k + 1]], add=True)

            @pl.when(k + 3 < NCHUNK)
            def _():
                pltpu.async_copy(tab.at[gidv.at[k + 3]], rows_b, sem_b)

        plsc.subcore_barrier()
        pltpu.sync_copy(acc.at[pl.ds(s * _SLICE, _SLICE)],
                        pp_hbm.at[c, p, pl.ds(s * _SLICE, _SLICE)])
        if with_deg and p == 0:
            pltpu.sync_copy(deg_sh.at[pl.ds(s * _SLICE, _SLICE)],
                            deg_hbm.at[c, pl.ds(s * _SLICE, _SLICE)])
        plsc.subcore_barrier()


def _edge(tabs, gid3, tid3, with_deg):
    out_type = [jax.ShapeDtypeStruct((NCORE, PASSES, NRP, PCOL), jnp.float32),
                jax.ShapeDtypeStruct((NCORE, NRP), jnp.float32)]
    return pl.kernel(
        functools.partial(_edge_kernel, with_deg),
        out_type=out_type,
        mesh=_MESH,
        compiler_params=pltpu.CompilerParams(use_tc_tiling_on_sc=False),
        scratch_types=[
            pltpu.VMEM((NCHUNK, CHUNK), jnp.int32),
            pltpu.VMEM((NCHUNK, CHUNK), jnp.int32),
            pltpu.VMEM((CHUNK, PCOL), jnp.float32),
            pltpu.VMEM((CHUNK, PCOL), jnp.float32),
            pltpu.VMEM((_ZROWS, PCOL), jnp.float32),
            pltpu.VMEM((_SLICE,), jnp.float32),
            pltpu.VMEM((CHUNK,), jnp.float32),
            pltpu.VMEM_SHARED((NRP, PCOL), jnp.float32),
            pltpu.VMEM_SHARED((NRP,), jnp.float32),
            pltpu.SemaphoreType.DMA,
            pltpu.SemaphoreType.DMA,
        ],
    )(tabs[0], tabs[1], tabs[2], tabs[3], gid3, tid3)


def kernel(x_paper, x_author, edge_index, edge_type,
           proj_w_paper, proj_b_paper, proj_w_author, proj_b_author,
           comp1, basis1, root1, comp2, basis2, root2):
    src = edge_index[0]
    dst = edge_index[1]

    gid, tid = _prep_ids(src, dst, edge_type)
    k = jnp.arange(EXTRA, dtype=jnp.int32)
    pad_gid = (k * 7919) % NR
    pad_tid = NR + k % (NRP - NR)
    gid3 = jnp.concatenate([gid.reshape(E), pad_gid]).reshape(
        NWORK, NCHUNK, CHUNK)
    tid3 = jnp.concatenate([tid.reshape(E), pad_tid]).reshape(
        NWORK, NCHUNK, CHUNK)

    xs = jnp.stack([x_paper, x_author])
    ws = jnp.stack([proj_w_paper, proj_w_author])
    bs = jnp.broadcast_to(jnp.stack([proj_b_paper, proj_b_author])[:, None, :],
                          (2, 8, HID))
    x = _proj(xs, ws, bs).reshape(N, HID)

    comps = jnp.stack([comp1, comp2])
    basisr = jnp.stack([basis1.reshape(NB, HID * HID),
                        basis2.reshape(NB, HID * HID)])
    wall = _wmix(comps, basisr)

    def wfull(l, root):
        wb = wall[l].reshape(R, HID, PASSES, PCOL).transpose(1, 2, 0, 3)
        return jnp.concatenate([root, wb.reshape(HID, R * HID)], axis=1)

    def layer(xin, root, l, relu, degx):
        o0, y0, y1, y2, y3 = _hmm(xin, wfull(l, root))
        tabs = [y.reshape(NR, PCOL) for y in (y0, y1, y2, y3)]
        pp, degp = _edge(tabs, gid3, tid3, with_deg=(l == 0))
        if degx is None:
            degx = jnp.repeat(degp.reshape(NCORE, NPAD, R), PCOL, axis=2)
        ppv = pp.reshape(NCORE, PASSES, NPAD, HID)
        y, cs, sq = _combine(ppv, o0, degx, relu)
        return _scale(y, cs, sq), degx

    h, degx = layer(x, root1, 0, relu=True, degx=None)
    out, _ = layer(h, root2, 1, relu=False, degx=degx)
    return (out, h)


# speedup vs baseline: 19.4228x; 1.3091x over previous
import functools

import jax
import jax.numpy as jnp
from jax import lax
from jax.experimental import pallas as pl
from jax.experimental.pallas import tpu as pltpu
from jax.experimental.pallas import tpu_sc as plsc

N = 10000
E = 320000
R = 4
NB = 8
HID = 128
EPS = 1e-5

NR = N * R
NRP = 40192
NPAD = NRP // R
NCORE = 2
NSUB = 16
NWORK = NCORE * NSUB
EPADDED = 327680
EXTRA = EPADDED - E
CHUNK = 128
NCHUNK = EPADDED // (NWORK * CHUNK)
PASSES = 4
PCOL = HID // PASSES
ROWBLK = 1000


def _prep_body(src_ref, dst_ref, et_ref, gid_ref, tid_ref):
    et = et_ref[...]
    gid_ref[...] = src_ref[...] * R + et
    tid_ref[...] = dst_ref[...] * R + et


def _prep_ids(src, dst, et):
    shp = jax.ShapeDtypeStruct((2500, HID), jnp.int32)
    return pl.pallas_call(
        _prep_body,
        out_shape=(shp, shp),
    )(src.reshape(2500, HID), dst.reshape(2500, HID), et.reshape(2500, HID))


def _wmix_body(comp_ref, basis_ref, w_ref):
    w_ref[0] = jnp.dot(comp_ref[0], basis_ref[0],
                       preferred_element_type=jnp.float32)


def _wmix(comp, basisr):
    return pl.pallas_call(
        _wmix_body,
        grid=(2,),
        in_specs=[
            pl.BlockSpec((1, R, NB), lambda g: (g, 0, 0)),
            pl.BlockSpec((1, NB, HID * HID), lambda g: (g, 0, 0)),
        ],
        out_specs=pl.BlockSpec((1, R, HID * HID), lambda g: (g, 0, 0)),
        out_shape=jax.ShapeDtypeStruct((2, R, HID * HID), jnp.float32),
    )(comp, basisr)


def _proj_body(x_ref, w_ref, b_ref, y_ref):
    y = jnp.dot(x_ref[0], w_ref[0], preferred_element_type=jnp.float32)
    y_ref[0] = jnp.maximum(y + b_ref[0, 0:1, :], 0.0)


def _proj(xs, ws, bs):
    nb = 5
    blk = 5000 // nb
    return pl.pallas_call(
        _proj_body,
        grid=(2, nb),
        in_specs=[
            pl.BlockSpec((1, blk, HID), lambda t, g: (t, g, 0)),
            pl.BlockSpec((1, HID, HID), lambda t, g: (t, 0, 0)),
            pl.BlockSpec((1, 8, HID), lambda t, g: (t, 0, 0)),
        ],
        out_specs=pl.BlockSpec((1, blk, HID), lambda t, g: (t, g, 0)),
        out_shape=jax.ShapeDtypeStruct((2, 5000, HID), jnp.float32),
    )(xs, ws, bs)


def _hmm_body(x_ref, w_ref, o0_ref, y0_ref, y1_ref, y2_ref, y3_ref):
    h = jnp.dot(x_ref[...], w_ref[...], preferred_element_type=jnp.float32)
    o0_ref[...] = h[:, 0:HID]
    y0_ref[...] = h[:, HID:2 * HID]
    y1_ref[...] = h[:, 2 * HID:3 * HID]
    y2_ref[...] = h[:, 3 * HID:4 * HID]
    y3_ref[...] = h[:, 4 * HID:5 * HID]


def _hmm(x, wfull):
    nb = N // ROWBLK
    shp = jax.ShapeDtypeStruct((N, HID), jnp.float32)
    return pl.pallas_call(
        _hmm_body,
        grid=(nb,),
        in_specs=[
            pl.BlockSpec((ROWBLK, HID), lambda g: (g, 0)),
            pl.BlockSpec((HID, 5 * HID), lambda g: (0, 0)),
        ],
        out_specs=[pl.BlockSpec((ROWBLK, HID), lambda g: (g, 0))] * 5,
        out_shape=(shp,) * 5,
    )(x, wfull)


def _combine_body(relu, pp_ref, o0_ref, deg_ref, y_ref, cs_ref, sq_ref):
    g = pl.program_id(0)
    d4 = 1.0 / jnp.maximum(deg_ref[0] + deg_ref[1], 1.0)
    rows = jax.lax.broadcasted_iota(jnp.int32, (R, HID), 0)
    cols = jax.lax.broadcasted_iota(jnp.int32, (R, HID), 1)
    sel = (cols // PCOL == rows).astype(jnp.float32)
    inv = jnp.dot(d4, sel, preferred_element_type=jnp.float32)
    parts = []
    for p in range(PASSES):
        w = (pp_ref[0, p] + pp_ref[1, p]) * inv
        t = w[:, 0:PCOL]
        for r in range(1, R):
            t = t + w[:, r * PCOL:(r + 1) * PCOL]
        parts.append(t)
    y = o0_ref[...] + jnp.concatenate(parts, axis=1)
    if relu:
        y = jnp.maximum(y, 0.0)
    y_ref[...] = y
    cs = jnp.broadcast_to(jnp.sum(y, axis=0, keepdims=True), (8, HID))
    sq = jnp.full((8, HID), jnp.sum(y * y))

    @pl.when(g == 0)
    def _():
        cs_ref[...] = cs
        sq_ref[...] = sq

    @pl.when(g > 0)
    def _():
        cs_ref[...] += cs
        sq_ref[...] += sq


def _combine(pp, out0, degx, relu):
    nb = N // ROWBLK
    small = jax.ShapeDtypeStruct((8, HID), jnp.float32)
    return pl.pallas_call(
        functools.partial(_combine_body, relu),
        grid=(nb,),
        in_specs=[
            pl.BlockSpec((2, PASSES, ROWBLK, HID), lambda g: (0, 0, g, 0)),
            pl.BlockSpec((ROWBLK, HID), lambda g: (g, 0)),
            pl.BlockSpec((2, ROWBLK, R), lambda g: (0, g, 0)),
        ],
        out_specs=[
            pl.BlockSpec((ROWBLK, HID), lambda g: (g, 0)),
            pl.BlockSpec((8, HID), lambda g: (0, 0)),
            pl.BlockSpec((8, HID), lambda g: (0, 0)),
        ],
        out_shape=(jax.ShapeDtypeStruct((N, HID), jnp.float32), small, small),
    )(pp, out0, degx)


def _scale_body(y_ref, cs_ref, sq_ref, o_ref):
    mu = cs_ref[0:1, :] * (1.0 / N)
    var = sq_ref[0, 0] * (1.0 / N) - jnp.sum(mu * mu)
    s = jax.lax.rsqrt(EPS + var)
    o_ref[...] = (y_ref[...] - mu) * s


def _scale(y, cs, sq):
    nb = N // ROWBLK
    return pl.pallas_call(
        _scale_body,
        grid=(nb,),
        in_specs=[
            pl.BlockSpec((ROWBLK, HID), lambda g: (g, 0)),
            pl.BlockSpec((8, HID), lambda g: (0, 0)),
            pl.BlockSpec((8, HID), lambda g: (0, 0)),
        ],
        out_specs=pl.BlockSpec((ROWBLK, HID), lambda g: (g, 0)),
        out_shape=jax.ShapeDtypeStruct((N, HID), jnp.float32),
    )(y, cs, sq)


_MESH = plsc.VectorSubcoreMesh(core_axis_name="c", subcore_axis_name="s",
                               num_cores=NCORE, num_subcores=NSUB)
_SLICE = NRP // NSUB


_ZROWS = 157


_NRING = 4


def _edge_kernel(with_deg, t0, t1, t2, t3, gid_hbm, tid_hbm, pp_hbm, deg_hbm,
                 gidv, tidv, rows, zbuf, acc, gsem, ssem,
                 zbuf1=None, ones_v=None, deg_sh=None):
    ci = lax.axis_index("c")
    s = lax.axis_index("s")
    wid = s * NCORE + ci
    tabs = (t0, t1, t2, t3)

    pltpu.sync_copy(gid_hbm.at[wid], gidv)
    pltpu.sync_copy(tid_hbm.at[wid], tidv)

    @pl.loop(0, _ZROWS)
    def _(i):
        zbuf[i, pl.ds(0, 16)] = jnp.zeros((16,), jnp.float32)
        zbuf[i, pl.ds(16, 16)] = jnp.zeros((16,), jnp.float32)
    if with_deg:
        @pl.loop(0, CHUNK // 16)
        def _(i):
            ones_v[pl.ds(i * 16, 16)] = jnp.ones((16,), jnp.float32)
        @pl.loop(0, _SLICE // 16)
        def _(i):
            zbuf1[pl.ds(i * 16, 16)] = jnp.zeros((16,), jnp.float32)
        pltpu.sync_copy(zbuf1, deg_sh.at[pl.ds(s * _SLICE, _SLICE)])
    del wid

    for p in range(PASSES):
        @pl.loop(0, _SLICE // _ZROWS)
        def _(j):
            pltpu.sync_copy(
                zbuf, acc.at[pl.ds(s * _SLICE + j * _ZROWS, _ZROWS)])
        plsc.subcore_barrier()

        tab = tabs[p]
        half = _NRING // 2

        for j in range(half):
            pltpu.async_copy(tab.at[gidv.at[j]], rows[j], gsem[j])

        @pl.loop(0, NCHUNK, step=_NRING)
        def _(k):
            for j in range(_NRING):
                c = k + j
                jj = (j + half) % _NRING
                @pl.when(c + half < NCHUNK)
                def _():
                    @pl.when(c >= half)
                    def _():
                        pltpu.make_async_copy(
                            rows[jj], acc.at[tidv.at[c - half]],
                            ssem[jj]).wait()
                    pltpu.async_copy(tab.at[gidv.at[c + half]],
                                     rows[jj], gsem[jj])

                pltpu.make_async_copy(tab.at[gidv.at[c]], rows[j],
                                      gsem[j]).wait()
                pltpu.async_copy(rows[j], acc.at[tidv.at[c]], ssem[j],
                                 add=True)
                if with_deg and p == 0:
                    pltpu.sync_copy(ones_v, deg_sh.at[tidv.at[c]], add=True)

        for j in range(_NRING):
            c = NCHUNK - _NRING + j
            pltpu.make_async_copy(rows[c % _NRING], acc.at[tidv.at[c]],
                                  ssem[c % _NRING]).wait()

        plsc.subcore_barrier()
        pltpu.sync_copy(acc.at[pl.ds(s * _SLICE, _SLICE)],
                        pp_hbm.at[ci, p, pl.ds(s * _SLICE, _SLICE)])
        if with_deg and p == 0:
            pltpu.sync_copy(deg_sh.at[pl.ds(s * _SLICE, _SLICE)],
                            deg_hbm.at[ci, pl.ds(s * _SLICE, _SLICE)])
        plsc.subcore_barrier()


def _edge(tabs, gid3, tid3, with_deg):
    out_type = [jax.ShapeDtypeStruct((NCORE, PASSES, NRP, PCOL), jnp.float32),
                jax.ShapeDtypeStruct((NCORE, NRP), jnp.float32)]
    return pl.kernel(
        functools.partial(_edge_kernel, with_deg),
        out_type=out_type,
        mesh=_MESH,
        compiler_params=pltpu.CompilerParams(use_tc_tiling_on_sc=False),
        scratch_types=[
            pltpu.VMEM((NCHUNK, CHUNK), jnp.int32),
            pltpu.VMEM((NCHUNK, CHUNK), jnp.int32),
            [pltpu.VMEM((CHUNK, PCOL), jnp.float32)] * _NRING,
            pltpu.VMEM((_ZROWS, PCOL), jnp.float32),
            pltpu.VMEM_SHARED((NRP, PCOL), jnp.float32),
            [pltpu.SemaphoreType.DMA] * _NRING,
            [pltpu.SemaphoreType.DMA] * _NRING,
        ] + ([
            pltpu.VMEM((_SLICE,), jnp.float32),
            pltpu.VMEM((CHUNK,), jnp.float32),
            pltpu.VMEM_SHARED((NRP,), jnp.float32),
        ] if with_deg else []),
    )(tabs[0], tabs[1], tabs[2], tabs[3], gid3, tid3)


def kernel(x_paper, x_author, edge_index, edge_type,
           proj_w_paper, proj_b_paper, proj_w_author, proj_b_author,
           comp1, basis1, root1, comp2, basis2, root2):
    src = edge_index[0]
    dst = edge_index[1]

    gid, tid = _prep_ids(src, dst, edge_type)
    k = jnp.arange(EXTRA, dtype=jnp.int32)
    pad_gid = (k * 7919) % NR
    pad_tid = NR + k % (NRP - NR)
    gid3 = jnp.concatenate([gid.reshape(E), pad_gid]).reshape(
        NWORK, NCHUNK, CHUNK)
    tid3 = jnp.concatenate([tid.reshape(E), pad_tid]).reshape(
        NWORK, NCHUNK, CHUNK)

    xs = jnp.stack([x_paper, x_author])
    ws = jnp.stack([proj_w_paper, proj_w_author])
    bs = jnp.broadcast_to(jnp.stack([proj_b_paper, proj_b_author])[:, None, :],
                          (2, 8, HID))
    x = _proj(xs, ws, bs).reshape(N, HID)

    comps = jnp.stack([comp1, comp2])
    basisr = jnp.stack([basis1.reshape(NB, HID * HID),
                        basis2.reshape(NB, HID * HID)])
    wall = _wmix(comps, basisr)

    def wfull(l, root):
        wb = wall[l].reshape(R, HID, PASSES, PCOL).transpose(1, 2, 0, 3)
        return jnp.concatenate([root, wb.reshape(HID, R * HID)], axis=1)

    def layer(xin, root, l, relu, degx):
        o0, y0, y1, y2, y3 = _hmm(xin, wfull(l, root))
        tabs = [y.reshape(NR, PCOL) for y in (y0, y1, y2, y3)]
        pp, degp = _edge(tabs, gid3, tid3, with_deg=(l == 0))
        if degx is None:
            degx = degp.reshape(NCORE, NPAD, R)
        ppv = pp.reshape(NCORE, PASSES, NPAD, HID)
        y, cs, sq = _combine(ppv, o0, degx, relu)
        return _scale(y, cs, sq), degx

    h, degx = layer(x, root1, 0, relu=True, degx=None)
    out, _ = layer(h, root2, 1, relu=False, degx=degx)
    return (out, h)


# speedup vs baseline: 20.0463x; 1.0321x over previous
import functools

import jax
import jax.numpy as jnp
from jax import lax
from jax.experimental import pallas as pl
from jax.experimental.pallas import tpu as pltpu
from jax.experimental.pallas import tpu_sc as plsc

N = 10000
E = 320000
R = 4
NB = 8
HID = 128
EPS = 1e-5

NR = N * R
NRP = 40192
NPAD = NRP // R
NCORE = 2
NSUB = 16
NWORK = NCORE * NSUB
EPADDED = 327680
EXTRA = EPADDED - E
CHUNK = 128
NCHUNK = EPADDED // (NWORK * CHUNK)
PASSES = 4
PCOL = HID // PASSES
ROWBLK = 1000


def _prep_body(src_ref, dst_ref, et_ref, gid_ref, tid_ref):
    et = et_ref[...]
    gid_ref[...] = src_ref[...] * R + et
    tid_ref[...] = dst_ref[...] * R + et


def _prep_ids(src, dst, et):
    shp = jax.ShapeDtypeStruct((2500, HID), jnp.int32)
    return pl.pallas_call(
        _prep_body,
        out_shape=(shp, shp),
    )(src.reshape(2500, HID), dst.reshape(2500, HID), et.reshape(2500, HID))


def _wmix_body(comp_ref, basis_ref, w_ref):
    w_ref[0] = jnp.dot(comp_ref[0], basis_ref[0],
                       preferred_element_type=jnp.float32)


def _wmix(comp, basisr):
    return pl.pallas_call(
        _wmix_body,
        grid=(2,),
        in_specs=[
            pl.BlockSpec((1, R, NB), lambda g: (g, 0, 0)),
            pl.BlockSpec((1, NB, HID * HID), lambda g: (g, 0, 0)),
        ],
        out_specs=pl.BlockSpec((1, R, HID * HID), lambda g: (g, 0, 0)),
        out_shape=jax.ShapeDtypeStruct((2, R, HID * HID), jnp.float32),
    )(comp, basisr)


def _proj_body(x_ref, w_ref, b_ref, y_ref):
    y = jnp.dot(x_ref[0], w_ref[0], preferred_element_type=jnp.float32)
    y_ref[0] = jnp.maximum(y + b_ref[0, 0:1, :], 0.0)


def _proj(xs, ws, bs):
    nb = 5
    blk = 5000 // nb
    return pl.pallas_call(
        _proj_body,
        grid=(2, nb),
        in_specs=[
            pl.BlockSpec((1, blk, HID), lambda t, g: (t, g, 0)),
            pl.BlockSpec((1, HID, HID), lambda t, g: (t, 0, 0)),
            pl.BlockSpec((1, 8, HID), lambda t, g: (t, 0, 0)),
        ],
        out_specs=pl.BlockSpec((1, blk, HID), lambda t, g: (t, g, 0)),
        out_shape=jax.ShapeDtypeStruct((2, 5000, HID), jnp.float32),
    )(xs, ws, bs)


def _hmm_body(x_ref, w_ref, y0_ref, y1_ref, y2_ref, y3_ref):
    h = jnp.dot(x_ref[...], w_ref[...], preferred_element_type=jnp.float32)
    y0_ref[...] = h[:, 0:HID]
    y1_ref[...] = h[:, HID:2 * HID]
    y2_ref[...] = h[:, 2 * HID:3 * HID]
    y3_ref[...] = h[:, 3 * HID:4 * HID]


def _hmm(x, wb):
    nb = N // ROWBLK
    shp = jax.ShapeDtypeStruct((N, HID), jnp.float32)
    return pl.pallas_call(
        _hmm_body,
        grid=(nb,),
        in_specs=[
            pl.BlockSpec((ROWBLK, HID), lambda g: (g, 0)),
            pl.BlockSpec((HID, 4 * HID), lambda g: (0, 0)),
        ],
        out_specs=[pl.BlockSpec((ROWBLK, HID), lambda g: (g, 0))] * 4,
        out_shape=(shp,) * 4,
    )(x, wb)


def _hmm_scale_body(y_ref, cs_ref, sq_ref, w_ref, h_ref, y0_ref, y1_ref,
                    y2_ref, y3_ref):
    mu = cs_ref[0:1, :] * (1.0 / N)
    var = sq_ref[0, 0] * (1.0 / N) - jnp.sum(mu * mu)
    sc = jax.lax.rsqrt(EPS + var)
    xb = (y_ref[...] - mu) * sc
    h_ref[...] = xb
    h = jnp.dot(xb, w_ref[...], preferred_element_type=jnp.float32)
    y0_ref[...] = h[:, 0:HID]
    y1_ref[...] = h[:, HID:2 * HID]
    y2_ref[...] = h[:, 2 * HID:3 * HID]
    y3_ref[...] = h[:, 3 * HID:4 * HID]


def _hmm_scale(y, cs, sq, wb):
    nb = N // ROWBLK
    shp = jax.ShapeDtypeStruct((N, HID), jnp.float32)
    return pl.pallas_call(
        _hmm_scale_body,
        grid=(nb,),
        in_specs=[
            pl.BlockSpec((ROWBLK, HID), lambda g: (g, 0)),
            pl.BlockSpec((8, HID), lambda g: (0, 0)),
            pl.BlockSpec((8, HID), lambda g: (0, 0)),
            pl.BlockSpec((HID, 4 * HID), lambda g: (0, 0)),
        ],
        out_specs=[pl.BlockSpec((ROWBLK, HID), lambda g: (g, 0))] * 5,
        out_shape=(shp,) * 5,
    )(y, cs, sq, wb)


def _combine_body(relu, pp_ref, x_ref, root_ref, deg_ref, y_ref, cs_ref,
                  sq_ref):
    g = pl.program_id(0)
    o0 = jnp.dot(x_ref[...], root_ref[...],
                 preferred_element_type=jnp.float32)
    d4 = 1.0 / jnp.maximum(deg_ref[0] + deg_ref[1], 1.0)
    rows = jax.lax.broadcasted_iota(jnp.int32, (R, HID), 0)
    cols = jax.lax.broadcasted_iota(jnp.int32, (R, HID), 1)
    sel = (cols // PCOL == rows).astype(jnp.float32)
    inv = jnp.dot(d4, sel, preferred_element_type=jnp.float32)
    parts = []
    for p in range(PASSES):
        w = (pp_ref[0, p] + pp_ref[1, p]) * inv
        t = w[:, 0:PCOL]
        for r in range(1, R):
            t = t + w[:, r * PCOL:(r + 1) * PCOL]
        parts.append(t)
    y = o0 + jnp.concatenate(parts, axis=1)
    if relu:
        y = jnp.maximum(y, 0.0)
    y_ref[...] = y
    cs = jnp.broadcast_to(jnp.sum(y, axis=0, keepdims=True), (8, HID))
    sq = jnp.full((8, HID), jnp.sum(y * y))

    @pl.when(g == 0)
    def _():
        cs_ref[...] = cs
        sq_ref[...] = sq

    @pl.when(g > 0)
    def _():
        cs_ref[...] += cs
        sq_ref[...] += sq


def _combine(pp, x, root, degx, relu):
    nb = N // ROWBLK
    small = jax.ShapeDtypeStruct((8, HID), jnp.float32)
    return pl.pallas_call(
        functools.partial(_combine_body, relu),
        grid=(nb,),
        in_specs=[
            pl.BlockSpec((2, PASSES, ROWBLK, HID), lambda g: (0, 0, g, 0)),
            pl.BlockSpec((ROWBLK, HID), lambda g: (g, 0)),
            pl.BlockSpec((HID, HID), lambda g: (0, 0)),
            pl.BlockSpec((2, ROWBLK, R), lambda g: (0, g, 0)),
        ],
        out_specs=[
            pl.BlockSpec((ROWBLK, HID), lambda g: (g, 0)),
            pl.BlockSpec((8, HID), lambda g: (0, 0)),
            pl.BlockSpec((8, HID), lambda g: (0, 0)),
        ],
        out_shape=(jax.ShapeDtypeStruct((N, HID), jnp.float32), small, small),
    )(pp, x, root, degx)


def _scale_body(y_ref, cs_ref, sq_ref, o_ref):
    mu = cs_ref[0:1, :] * (1.0 / N)
    var = sq_ref[0, 0] * (1.0 / N) - jnp.sum(mu * mu)
    s = jax.lax.rsqrt(EPS + var)
    o_ref[...] = (y_ref[...] - mu) * s


def _scale(y, cs, sq):
    nb = N // ROWBLK
    return pl.pallas_call(
        _scale_body,
        grid=(nb,),
        in_specs=[
            pl.BlockSpec((ROWBLK, HID), lambda g: (g, 0)),
            pl.BlockSpec((8, HID), lambda g: (0, 0)),
            pl.BlockSpec((8, HID), lambda g: (0, 0)),
        ],
        out_specs=pl.BlockSpec((ROWBLK, HID), lambda g: (g, 0)),
        out_shape=jax.ShapeDtypeStruct((N, HID), jnp.float32),
    )(y, cs, sq)


_MESH = plsc.VectorSubcoreMesh(core_axis_name="c", subcore_axis_name="s",
                               num_cores=NCORE, num_subcores=NSUB)
_SLICE = NRP // NSUB


_ZROWS = 157


_NRING = 4


def _edge_kernel(with_deg, t0, t1, t2, t3, gid_hbm, tid_hbm, pp_hbm, deg_hbm,
                 gidv, tidv, rows, zbuf, acc, gsem, ssem,
                 zbuf1=None, ones_v=None, deg_sh=None):
    ci = lax.axis_index("c")
    s = lax.axis_index("s")
    wid = s * NCORE + ci
    tabs = (t0, t1, t2, t3)

    pltpu.sync_copy(gid_hbm.at[wid], gidv)
    pltpu.sync_copy(tid_hbm.at[wid], tidv)

    @pl.loop(0, _ZROWS)
    def _(i):
        zbuf[i, pl.ds(0, 16)] = jnp.zeros((16,), jnp.float32)
        zbuf[i, pl.ds(16, 16)] = jnp.zeros((16,), jnp.float32)
    if with_deg:
        @pl.loop(0, CHUNK // 16)
        def _(i):
            ones_v[pl.ds(i * 16, 16)] = jnp.ones((16,), jnp.float32)
        @pl.loop(0, _SLICE // 16)
        def _(i):
            zbuf1[pl.ds(i * 16, 16)] = jnp.zeros((16,), jnp.float32)
        pltpu.sync_copy(zbuf1, deg_sh.at[pl.ds(s * _SLICE, _SLICE)])
    del wid

    @pl.loop(0, _SLICE // _ZROWS)
    def _(j):
        pltpu.sync_copy(
            zbuf, acc.at[pl.ds(s * _SLICE + j * _ZROWS, _ZROWS)])

    for p in range(PASSES):
        tab = tabs[p]
        half = _NRING // 2

        for j in range(half):
            pltpu.async_copy(tab.at[gidv.at[j]], rows[j], gsem[j])
        plsc.subcore_barrier()

        @pl.loop(0, NCHUNK, step=_NRING)
        def _(k):
            for j in range(_NRING):
                c = k + j
                jj = (j + half) % _NRING
                @pl.when(c + half < NCHUNK)
                def _():
                    @pl.when(c >= half)
                    def _():
                        pltpu.make_async_copy(
                            rows[jj], acc.at[tidv.at[c - half]],
                            ssem[jj]).wait()
                    pltpu.async_copy(tab.at[gidv.at[c + half]],
                                     rows[jj], gsem[jj])

                pltpu.make_async_copy(tab.at[gidv.at[c]], rows[j],
                                      gsem[j]).wait()
                pltpu.async_copy(rows[j], acc.at[tidv.at[c]], ssem[j],
                                 add=True)
                if with_deg and p == 0:
                    pltpu.sync_copy(ones_v, deg_sh.at[tidv.at[c]], add=True)

        for j in range(_NRING):
            c = NCHUNK - _NRING + j
            pltpu.make_async_copy(rows[c % _NRING], acc.at[tidv.at[c]],
                                  ssem[c % _NRING]).wait()

        plsc.subcore_barrier()
        pltpu.sync_copy(acc.at[pl.ds(s * _SLICE, _SLICE)],
                        pp_hbm.at[ci, p, pl.ds(s * _SLICE, _SLICE)])
        if with_deg and p == 0:
            pltpu.sync_copy(deg_sh.at[pl.ds(s * _SLICE, _SLICE)],
                            deg_hbm.at[ci, pl.ds(s * _SLICE, _SLICE)])
        if p < PASSES - 1:
            @pl.loop(0, _SLICE // _ZROWS)
            def _(j):
                pltpu.sync_copy(
                    zbuf, acc.at[pl.ds(s * _SLICE + j * _ZROWS, _ZROWS)])


def _edge(tabs, gid3, tid3, with_deg):
    out_type = [jax.ShapeDtypeStruct((NCORE, PASSES, NRP, PCOL), jnp.float32),
                jax.ShapeDtypeStruct((NCORE, NRP), jnp.float32)]
    return pl.kernel(
        functools.partial(_edge_kernel, with_deg),
        out_type=out_type,
        mesh=_MESH,
        compiler_params=pltpu.CompilerParams(use_tc_tiling_on_sc=False),
        scratch_types=[
            pltpu.VMEM((NCHUNK, CHUNK), jnp.int32),
            pltpu.VMEM((NCHUNK, CHUNK), jnp.int32),
            [pltpu.VMEM((CHUNK, PCOL), jnp.float32)] * _NRING,
            pltpu.VMEM((_ZROWS, PCOL), jnp.float32),
            pltpu.VMEM_SHARED((NRP, PCOL), jnp.float32),
            [pltpu.SemaphoreType.DMA] * _NRING,
            [pltpu.SemaphoreType.DMA] * _NRING,
        ] + ([
            pltpu.VMEM((_SLICE,), jnp.float32),
            pltpu.VMEM((CHUNK,), jnp.float32),
            pltpu.VMEM_SHARED((NRP,), jnp.float32),
        ] if with_deg else []),
    )(tabs[0], tabs[1], tabs[2], tabs[3], gid3, tid3)


def kernel(x_paper, x_author, edge_index, edge_type,
           proj_w_paper, proj_b_paper, proj_w_author, proj_b_author,
           comp1, basis1, root1, comp2, basis2, root2):
    src = edge_index[0]
    dst = edge_index[1]

    gid, tid = _prep_ids(src, dst, edge_type)
    k = jnp.arange(EXTRA, dtype=jnp.int32)
    pad_gid = (k * 7919) % NR
    pad_tid = NR + k % (NRP - NR)
    gid3 = jnp.concatenate([gid.reshape(E), pad_gid]).reshape(
        NWORK, NCHUNK, CHUNK)
    tid3 = jnp.concatenate([tid.reshape(E), pad_tid]).reshape(
        NWORK, NCHUNK, CHUNK)

    xs = jnp.stack([x_paper, x_author])
    ws = jnp.stack([proj_w_paper, proj_w_author])
    bs = jnp.broadcast_to(jnp.stack([proj_b_paper, proj_b_author])[:, None, :],
                          (2, 8, HID))
    x = _proj(xs, ws, bs).reshape(N, HID)

    comps = jnp.stack([comp1, comp2])
    basisr = jnp.stack([basis1.reshape(NB, HID * HID),
                        basis2.reshape(NB, HID * HID)])
    wall = _wmix(comps, basisr)

    def wb(l):
        w = wall[l].reshape(R, HID, PASSES, PCOL).transpose(1, 2, 0, 3)
        return w.reshape(HID, R * HID)

    ys = _hmm(x, wb(0))
    tabs = [y.reshape(NR, PCOL) for y in ys]
    pp, degp = _edge(tabs, gid3, tid3, with_deg=True)
    degx = degp.reshape(NCORE, NPAD, R)
    y1_, cs1, sq1 = _combine(pp.reshape(NCORE, PASSES, NPAD, HID),
                             x, root1, degx, relu=True)

    h, t0, t1, t2, t3 = _hmm_scale(y1_, cs1, sq1, wb(1))
    tabs2 = [t.reshape(NR, PCOL) for t in (t0, t1, t2, t3)]
    pp2, _ = _edge(tabs2, gid3, tid3, with_deg=False)
    y2_, cs2, sq2 = _combine(pp2.reshape(NCORE, PASSES, NPAD, HID),
                             h, root2, degx, relu=False)
    out = _scale(y2_, cs2, sq2)
    return (out, h)


# speedup vs baseline: 20.5920x; 1.0272x over previous
import functools

import jax
import jax.numpy as jnp
from jax import lax
from jax.experimental import pallas as pl
from jax.experimental.pallas import tpu as pltpu
from jax.experimental.pallas import tpu_sc as plsc

N = 10000
E = 320000
R = 4
NB = 8
HID = 128
EPS = 1e-5

NR = N * R
NRP = 40192
NPAD = NRP // R
NCORE = 2
NSUB = 16
NWORK = NCORE * NSUB
EPADDED = 327680
EXTRA = EPADDED - E
CHUNK = 128
NCHUNK = EPADDED // (NWORK * CHUNK)
PASSES = 4
PCOL = HID // PASSES
ROWBLK = 2000


def _prep_body(src_ref, dst_ref, et_ref, gid_ref, tid_ref):
    et = et_ref[...]
    gid_ref[...] = src_ref[...] * R + et
    tid_ref[...] = dst_ref[...] * R + et


def _prep_ids(src, dst, et):
    shp = jax.ShapeDtypeStruct((2500, HID), jnp.int32)
    return pl.pallas_call(
        _prep_body,
        out_shape=(shp, shp),
    )(src.reshape(2500, HID), dst.reshape(2500, HID), et.reshape(2500, HID))


def _wmix_body(comp_ref, basis_ref, w_ref):
    w_ref[0] = jnp.dot(comp_ref[0], basis_ref[0],
                       preferred_element_type=jnp.float32)


def _wmix(comp, basisr):
    return pl.pallas_call(
        _wmix_body,
        grid=(2,),
        in_specs=[
            pl.BlockSpec((1, R, NB), lambda g: (g, 0, 0)),
            pl.BlockSpec((1, NB, HID * HID), lambda g: (g, 0, 0)),
        ],
        out_specs=pl.BlockSpec((1, R, HID * HID), lambda g: (g, 0, 0)),
        out_shape=jax.ShapeDtypeStruct((2, R, HID * HID), jnp.float32),
    )(comp, basisr)


def _proj_body(x_ref, w_ref, b_ref, y_ref):
    y = jnp.dot(x_ref[0], w_ref[0], preferred_element_type=jnp.float32)
    y_ref[0] = jnp.maximum(y + b_ref[0, 0:1, :], 0.0)


def _proj(xs, ws, bs):
    nb = 5
    blk = 5000 // nb
    return pl.pallas_call(
        _proj_body,
        grid=(2, nb),
        in_specs=[
            pl.BlockSpec((1, blk, HID), lambda t, g: (t, g, 0)),
            pl.BlockSpec((1, HID, HID), lambda t, g: (t, 0, 0)),
            pl.BlockSpec((1, 8, HID), lambda t, g: (t, 0, 0)),
        ],
        out_specs=pl.BlockSpec((1, blk, HID), lambda t, g: (t, g, 0)),
        out_shape=jax.ShapeDtypeStruct((2, 5000, HID), jnp.float32),
    )(xs, ws, bs)


def _hmm_body(x_ref, w_ref, y0_ref, y1_ref, y2_ref, y3_ref):
    h = jnp.dot(x_ref[...], w_ref[...], preferred_element_type=jnp.float32)
    y0_ref[...] = h[:, 0:HID]
    y1_ref[...] = h[:, HID:2 * HID]
    y2_ref[...] = h[:, 2 * HID:3 * HID]
    y3_ref[...] = h[:, 3 * HID:4 * HID]


def _hmm(x, wb):
    nb = N // ROWBLK
    shp = jax.ShapeDtypeStruct((N, HID), jnp.float32)
    return pl.pallas_call(
        _hmm_body,
        grid=(nb,),
        in_specs=[
            pl.BlockSpec((ROWBLK, HID), lambda g: (g, 0)),
            pl.BlockSpec((HID, 4 * HID), lambda g: (0, 0)),
        ],
        out_specs=[pl.BlockSpec((ROWBLK, HID), lambda g: (g, 0))] * 4,
        out_shape=(shp,) * 4,
    )(x, wb)


def _hmm_scale_body(y_ref, cs_ref, sq_ref, w_ref, h_ref, y0_ref, y1_ref,
                    y2_ref, y3_ref):
    mu = cs_ref[0:1, :] * (1.0 / N)
    var = sq_ref[0, 0] * (1.0 / N) - jnp.sum(mu * mu)
    sc = jax.lax.rsqrt(EPS + var)
    xb = (y_ref[...] - mu) * sc
    h_ref[...] = xb
    h = jnp.dot(xb, w_ref[...], preferred_element_type=jnp.float32)
    y0_ref[...] = h[:, 0:HID]
    y1_ref[...] = h[:, HID:2 * HID]
    y2_ref[...] = h[:, 2 * HID:3 * HID]
    y3_ref[...] = h[:, 3 * HID:4 * HID]


def _hmm_scale(y, cs, sq, wb):
    nb = N // ROWBLK
    shp = jax.ShapeDtypeStruct((N, HID), jnp.float32)
    return pl.pallas_call(
        _hmm_scale_body,
        grid=(nb,),
        in_specs=[
            pl.BlockSpec((ROWBLK, HID), lambda g: (g, 0)),
            pl.BlockSpec((8, HID), lambda g: (0, 0)),
            pl.BlockSpec((8, HID), lambda g: (0, 0)),
            pl.BlockSpec((HID, 4 * HID), lambda g: (0, 0)),
        ],
        out_specs=[pl.BlockSpec((ROWBLK, HID), lambda g: (g, 0))] * 5,
        out_shape=(shp,) * 5,
    )(y, cs, sq, wb)


def _combine_body(relu, pp_ref, x_ref, root_ref, deg_ref, y_ref, cs_ref,
                  sq_ref):
    g = pl.program_id(0)
    o0 = jnp.dot(x_ref[...], root_ref[...],
                 preferred_element_type=jnp.float32)
    d4 = 1.0 / jnp.maximum(deg_ref[0] + deg_ref[1], 1.0)
    rows = jax.lax.broadcasted_iota(jnp.int32, (R, HID), 0)
    cols = jax.lax.broadcasted_iota(jnp.int32, (R, HID), 1)
    sel = (cols // PCOL == rows).astype(jnp.float32)
    inv = jnp.dot(d4, sel, preferred_element_type=jnp.float32)
    parts = []
    for p in range(PASSES):
        w = (pp_ref[0, p] + pp_ref[1, p]) * inv
        t = w[:, 0:PCOL]
        for r in range(1, R):
            t = t + w[:, r * PCOL:(r + 1) * PCOL]
        parts.append(t)
    y = o0 + jnp.concatenate(parts, axis=1)
    if relu:
        y = jnp.maximum(y, 0.0)
    y_ref[...] = y
    cs = jnp.broadcast_to(jnp.sum(y, axis=0, keepdims=True), (8, HID))
    sq = jnp.full((8, HID), jnp.sum(y * y))

    @pl.when(g == 0)
    def _():
        cs_ref[...] = cs
        sq_ref[...] = sq

    @pl.when(g > 0)
    def _():
        cs_ref[...] += cs
        sq_ref[...] += sq


def _combine(pp, x, root, degx, relu):
    nb = N // ROWBLK
    small = jax.ShapeDtypeStruct((8, HID), jnp.float32)
    return pl.pallas_call(
        functools.partial(_combine_body, relu),
        grid=(nb,),
        in_specs=[
            pl.BlockSpec((2, PASSES, ROWBLK, HID), lambda g: (0, 0, g, 0)),
            pl.BlockSpec((ROWBLK, HID), lambda g: (g, 0)),
            pl.BlockSpec((HID, HID), lambda g: (0, 0)),
            pl.BlockSpec((2, ROWBLK, R), lambda g: (0, g, 0)),
        ],
        out_specs=[
            pl.BlockSpec((ROWBLK, HID), lambda g: (g, 0)),
            pl.BlockSpec((8, HID), lambda g: (0, 0)),
            pl.BlockSpec((8, HID), lambda g: (0, 0)),
        ],
        out_shape=(jax.ShapeDtypeStruct((N, HID), jnp.float32), small, small),
    )(pp, x, root, degx)


def _scale_body(y_ref, cs_ref, sq_ref, o_ref):
    mu = cs_ref[0:1, :] * (1.0 / N)
    var = sq_ref[0, 0] * (1.0 / N) - jnp.sum(mu * mu)
    s = jax.lax.rsqrt(EPS + var)
    o_ref[...] = (y_ref[...] - mu) * s


def _scale(y, cs, sq):
    nb = N // ROWBLK
    return pl.pallas_call(
        _scale_body,
        grid=(nb,),
        in_specs=[
            pl.BlockSpec((ROWBLK, HID), lambda g: (g, 0)),
            pl.BlockSpec((8, HID), lambda g: (0, 0)),
            pl.BlockSpec((8, HID), lambda g: (0, 0)),
        ],
        out_specs=pl.BlockSpec((ROWBLK, HID), lambda g: (g, 0)),
        out_shape=jax.ShapeDtypeStruct((N, HID), jnp.float32),
    )(y, cs, sq)


_MESH = plsc.VectorSubcoreMesh(core_axis_name="c", subcore_axis_name="s",
                               num_cores=NCORE, num_subcores=NSUB)
_SLICE = NRP // NSUB


_ZROWS = 157


_NRING = 4


def _edge_kernel(with_deg, t0, t1, t2, t3, gid_hbm, tid_hbm, pp_hbm, deg_hbm,
                 gidv, tidv, rows, zbuf, acc, gsem, ssem,
                 zbuf1=None, ones_v=None, deg_sh=None, dsem=None):
    ci = lax.axis_index("c")
    s = lax.axis_index("s")
    wid = s * NCORE + ci
    tabs = (t0, t1, t2, t3)

    pltpu.sync_copy(gid_hbm.at[wid], gidv)
    pltpu.sync_copy(tid_hbm.at[wid], tidv)

    @pl.loop(0, _ZROWS)
    def _(i):
        zbuf[i, pl.ds(0, 16)] = jnp.zeros((16,), jnp.float32)
        zbuf[i, pl.ds(16, 16)] = jnp.zeros((16,), jnp.float32)
    if with_deg:
        @pl.loop(0, CHUNK // 16)
        def _(i):
            ones_v[pl.ds(i * 16, 16)] = jnp.ones((16,), jnp.float32)
        @pl.loop(0, _SLICE // 16)
        def _(i):
            zbuf1[pl.ds(i * 16, 16)] = jnp.zeros((16,), jnp.float32)
        pltpu.sync_copy(zbuf1, deg_sh.at[pl.ds(s * _SLICE, _SLICE)])
    del wid

    @pl.loop(0, _SLICE // _ZROWS)
    def _(j):
        pltpu.sync_copy(
            zbuf, acc.at[pl.ds(s * _SLICE + j * _ZROWS, _ZROWS)])

    for p in range(PASSES):
        tab = tabs[p]
        half = _NRING // 2

        for j in range(half):
            pltpu.async_copy(tab.at[gidv.at[j]], rows[j], gsem[j])
        plsc.subcore_barrier()

        @pl.loop(0, NCHUNK, step=_NRING)
        def _(k):
            for j in range(_NRING):
                c = k + j
                jj = (j + half) % _NRING
                @pl.when(c + half < NCHUNK)
                def _():
                    @pl.when(c >= half)
                    def _():
                        pltpu.make_async_copy(
                            rows[jj], acc.at[tidv.at[c - half]],
                            ssem[jj]).wait()
                    pltpu.async_copy(tab.at[gidv.at[c + half]],
                                     rows[jj], gsem[jj])

                pltpu.make_async_copy(tab.at[gidv.at[c]], rows[j],
                                      gsem[j]).wait()
                pltpu.async_copy(rows[j], acc.at[tidv.at[c]], ssem[j],
                                 add=True)
                if with_deg and p == 0:
                    @pl.when(c >= 8)
                    def _():
                        pltpu.make_async_copy(ones_v, deg_sh.at[tidv.at[c - 8]],
                                              dsem).wait()
                    pltpu.async_copy(ones_v, deg_sh.at[tidv.at[c]], dsem,
                                     add=True)

        for j in range(_NRING):
            c = NCHUNK - _NRING + j
            pltpu.make_async_copy(rows[c % _NRING], acc.at[tidv.at[c]],
                                  ssem[c % _NRING]).wait()
        if with_deg and p == 0:
            @pl.loop(NCHUNK - 8, NCHUNK)
            def _(k):
                pltpu.make_async_copy(ones_v, deg_sh.at[tidv.at[k]],
                                      dsem).wait()

        plsc.subcore_barrier()
        pltpu.sync_copy(acc.at[pl.ds(s * _SLICE, _SLICE)],
                        pp_hbm.at[ci, p, pl.ds(s * _SLICE, _SLICE)])
        if with_deg and p == 0:
            pltpu.sync_copy(deg_sh.at[pl.ds(s * _SLICE, _SLICE)],
                            deg_hbm.at[ci, pl.ds(s * _SLICE, _SLICE)])
        if p < PASSES - 1:
            @pl.loop(0, _SLICE // _ZROWS)
            def _(j):
                pltpu.sync_copy(
                    zbuf, acc.at[pl.ds(s * _SLICE + j * _ZROWS, _ZROWS)])


def _edge(tabs, gid3, tid3, with_deg):
    out_type = [jax.ShapeDtypeStruct((NCORE, PASSES, NRP, PCOL), jnp.float32),
                jax.ShapeDtypeStruct((NCORE, NRP), jnp.float32)]
    return pl.kernel(
        functools.partial(_edge_kernel, with_deg),
        out_type=out_type,
        mesh=_MESH,
        compiler_params=pltpu.CompilerParams(use_tc_tiling_on_sc=False),
        scratch_types=[
            pltpu.VMEM((NCHUNK, CHUNK), jnp.int32),
            pltpu.VMEM((NCHUNK, CHUNK), jnp.int32),
            [pltpu.VMEM((CHUNK, PCOL), jnp.float32)] * _NRING,
            pltpu.VMEM((_ZROWS, PCOL), jnp.float32),
            pltpu.VMEM_SHARED((NRP, PCOL), jnp.float32),
            [pltpu.SemaphoreType.DMA] * _NRING,
            [pltpu.SemaphoreType.DMA] * _NRING,
        ] + ([
            pltpu.VMEM((_SLICE,), jnp.float32),
            pltpu.VMEM((CHUNK,), jnp.float32),
            pltpu.VMEM_SHARED((NRP,), jnp.float32),
            pltpu.SemaphoreType.DMA,
        ] if with_deg else []),
    )(tabs[0], tabs[1], tabs[2], tabs[3], gid3, tid3)


def kernel(x_paper, x_author, edge_index, edge_type,
           proj_w_paper, proj_b_paper, proj_w_author, proj_b_author,
           comp1, basis1, root1, comp2, basis2, root2):
    src = edge_index[0]
    dst = edge_index[1]

    gid, tid = _prep_ids(src, dst, edge_type)
    k = jnp.arange(EXTRA, dtype=jnp.int32)
    pad_gid = (k * 7919) % NR
    pad_tid = NR + k % (NRP - NR)
    gid3 = jnp.concatenate([gid.reshape(E), pad_gid]).reshape(
        NWORK, NCHUNK, CHUNK)
    tid3 = jnp.concatenate([tid.reshape(E), pad_tid]).reshape(
        NWORK, NCHUNK, CHUNK)

    xs = jnp.stack([x_paper, x_author])
    ws = jnp.stack([proj_w_paper, proj_w_author])
    bs = jnp.broadcast_to(jnp.stack([proj_b_paper, proj_b_author])[:, None, :],
                          (2, 8, HID))
    x = _proj(xs, ws, bs).reshape(N, HID)

    comps = jnp.stack([comp1, comp2])
    basisr = jnp.stack([basis1.reshape(NB, HID * HID),
                        basis2.reshape(NB, HID * HID)])
    wall = _wmix(comps, basisr)

    def wb(l):
        w = wall[l].reshape(R, HID, PASSES, PCOL).transpose(1, 2, 0, 3)
        return w.reshape(HID, R * HID)

    ys = _hmm(x, wb(0))
    tabs = [y.reshape(NR, PCOL) for y in ys]
    pp, degp = _edge(tabs, gid3, tid3, with_deg=True)
    degx = degp.reshape(NCORE, NPAD, R)
    y1_, cs1, sq1 = _combine(pp.reshape(NCORE, PASSES, NPAD, HID),
                             x, root1, degx, relu=True)

    h, t0, t1, t2, t3 = _hmm_scale(y1_, cs1, sq1, wb(1))
    tabs2 = [t.reshape(NR, PCOL) for t in (t0, t1, t2, t3)]
    pp2, _ = _edge(tabs2, gid3, tid3, with_deg=False)
    y2_, cs2, sq2 = _combine(pp2.reshape(NCORE, PASSES, NPAD, HID),
                             h, root2, degx, relu=False)
    out = _scale(y2_, cs2, sq2)
    return (out, h)


# speedup vs baseline: 20.6143x; 1.0011x over previous
import functools

import jax
import jax.numpy as jnp
from jax import lax
from jax.experimental import pallas as pl
from jax.experimental.pallas import tpu as pltpu
from jax.experimental.pallas import tpu_sc as plsc

N = 10000
E = 320000
R = 4
NB = 8
HID = 128
EPS = 1e-5

NR = N * R
NRP = 40192
NPAD = NRP // R
NCORE = 2
NSUB = 16
NWORK = NCORE * NSUB
EPADDED = 327680
EXTRA = EPADDED - E
CHUNK = 128
NCHUNK = EPADDED // (NWORK * CHUNK)
PASSES = 4
PCOL = HID // PASSES
ROWBLK = 2000


def _prep_body(src_ref, dst_ref, et_ref, gid_ref, tid_ref):
    et = et_ref[...]
    gid_ref[...] = src_ref[...] * R + et
    tid_ref[...] = dst_ref[...] * R + et


def _prep_ids(src, dst, et):
    shp = jax.ShapeDtypeStruct((2500, HID), jnp.int32)
    return pl.pallas_call(
        _prep_body,
        out_shape=(shp, shp),
    )(src.reshape(2500, HID), dst.reshape(2500, HID), et.reshape(2500, HID))


def _wmix_body(comp_ref, basis_ref, w_ref):
    w_ref[0] = jnp.dot(comp_ref[0], basis_ref[0],
                       preferred_element_type=jnp.float32)


def _wmix(comp, basisr):
    return pl.pallas_call(
        _wmix_body,
        grid=(2,),
        in_specs=[
            pl.BlockSpec((1, R, NB), lambda g: (g, 0, 0)),
            pl.BlockSpec((1, NB, HID * HID), lambda g: (g, 0, 0)),
        ],
        out_specs=pl.BlockSpec((1, R, HID * HID), lambda g: (g, 0, 0)),
        out_shape=jax.ShapeDtypeStruct((2, R, HID * HID), jnp.float32),
    )(comp, basisr)


def _proj_body(x_ref, w_ref, b_ref, y_ref):
    y = jnp.dot(x_ref[0], w_ref[0], preferred_element_type=jnp.float32)
    y_ref[0] = jnp.maximum(y + b_ref[0, 0:1, :], 0.0)


def _proj(xs, ws, bs):
    nb = 5
    blk = 5000 // nb
    return pl.pallas_call(
        _proj_body,
        grid=(2, nb),
        in_specs=[
            pl.BlockSpec((1, blk, HID), lambda t, g: (t, g, 0)),
            pl.BlockSpec((1, HID, HID), lambda t, g: (t, 0, 0)),
            pl.BlockSpec((1, 8, HID), lambda t, g: (t, 0, 0)),
        ],
        out_specs=pl.BlockSpec((1, blk, HID), lambda t, g: (t, g, 0)),
        out_shape=jax.ShapeDtypeStruct((2, 5000, HID), jnp.float32),
    )(xs, ws, bs)


def _hmm_body(x_ref, w_ref, y0_ref, y1_ref, y2_ref, y3_ref):
    h = jnp.dot(x_ref[...], w_ref[...], preferred_element_type=jnp.float32)
    y0_ref[...] = h[:, 0:HID]
    y1_ref[...] = h[:, HID:2 * HID]
    y2_ref[...] = h[:, 2 * HID:3 * HID]
    y3_ref[...] = h[:, 3 * HID:4 * HID]


def _hmm(x, wb):
    nb = N // ROWBLK
    shp = jax.ShapeDtypeStruct((N, HID), jnp.float32)
    return pl.pallas_call(
        _hmm_body,
        grid=(nb,),
        in_specs=[
            pl.BlockSpec((ROWBLK, HID), lambda g: (g, 0)),
            pl.BlockSpec((HID, 4 * HID), lambda g: (0, 0)),
        ],
        out_specs=[pl.BlockSpec((ROWBLK, HID), lambda g: (g, 0))] * 4,
        out_shape=(shp,) * 4,
    )(x, wb)


def _hmm_scale_body(y_ref, cs_ref, sq_ref, w_ref, h_ref, y0_ref, y1_ref,
                    y2_ref, y3_ref):
    mu = cs_ref[0:1, :] * (1.0 / N)
    var = sq_ref[0, 0] * (1.0 / N) - jnp.sum(mu * mu)
    sc = jax.lax.rsqrt(EPS + var)
    xb = (y_ref[...] - mu) * sc
    h_ref[...] = xb
    h = jnp.dot(xb, w_ref[...], preferred_element_type=jnp.float32)
    y0_ref[...] = h[:, 0:HID]
    y1_ref[...] = h[:, HID:2 * HID]
    y2_ref[...] = h[:, 2 * HID:3 * HID]
    y3_ref[...] = h[:, 3 * HID:4 * HID]


def _hmm_scale(y, cs, sq, wb):
    nb = N // ROWBLK
    shp = jax.ShapeDtypeStruct((N, HID), jnp.float32)
    return pl.pallas_call(
        _hmm_scale_body,
        grid=(nb,),
        in_specs=[
            pl.BlockSpec((ROWBLK, HID), lambda g: (g, 0)),
            pl.BlockSpec((8, HID), lambda g: (0, 0)),
            pl.BlockSpec((8, HID), lambda g: (0, 0)),
            pl.BlockSpec((HID, 4 * HID), lambda g: (0, 0)),
        ],
        out_specs=[pl.BlockSpec((ROWBLK, HID), lambda g: (g, 0))] * 5,
        out_shape=(shp,) * 5,
    )(y, cs, sq, wb)


def _combine_body(relu, pp_ref, x_ref, root_ref, deg_ref, y_ref, cs_ref,
                  sq_ref):
    g = pl.program_id(0)
    o0 = jnp.dot(x_ref[...], root_ref[...],
                 preferred_element_type=jnp.float32)
    d4 = 1.0 / jnp.maximum(deg_ref[0] + deg_ref[1], 1.0)
    rows = jax.lax.broadcasted_iota(jnp.int32, (R, HID), 0)
    cols = jax.lax.broadcasted_iota(jnp.int32, (R, HID), 1)
    sel = (cols // PCOL == rows).astype(jnp.float32)
    inv = jnp.dot(d4, sel, preferred_element_type=jnp.float32)
    parts = []
    for p in range(PASSES):
        w = (pp_ref[0, p] + pp_ref[1, p]) * inv
        t = w[:, 0:PCOL]
        for r in range(1, R):
            t = t + w[:, r * PCOL:(r + 1) * PCOL]
        parts.append(t)
    y = o0 + jnp.concatenate(parts, axis=1)
    if relu:
        y = jnp.maximum(y, 0.0)
    y_ref[...] = y
    cs = jnp.broadcast_to(jnp.sum(y, axis=0, keepdims=True), (8, HID))
    sq = jnp.full((8, HID), jnp.sum(y * y))

    @pl.when(g == 0)
    def _():
        cs_ref[...] = cs
        sq_ref[...] = sq

    @pl.when(g > 0)
    def _():
        cs_ref[...] += cs
        sq_ref[...] += sq


def _combine(pp, x, root, degx, relu):
    nb = N // ROWBLK
    small = jax.ShapeDtypeStruct((8, HID), jnp.float32)
    return pl.pallas_call(
        functools.partial(_combine_body, relu),
        grid=(nb,),
        in_specs=[
            pl.BlockSpec((2, PASSES, ROWBLK, HID), lambda g: (0, 0, g, 0)),
            pl.BlockSpec((ROWBLK, HID), lambda g: (g, 0)),
            pl.BlockSpec((HID, HID), lambda g: (0, 0)),
            pl.BlockSpec((2, ROWBLK, R), lambda g: (0, g, 0)),
        ],
        out_specs=[
            pl.BlockSpec((ROWBLK, HID), lambda g: (g, 0)),
            pl.BlockSpec((8, HID), lambda g: (0, 0)),
            pl.BlockSpec((8, HID), lambda g: (0, 0)),
        ],
        out_shape=(jax.ShapeDtypeStruct((N, HID), jnp.float32), small, small),
    )(pp, x, root, degx)


def _scale_body(y_ref, cs_ref, sq_ref, o_ref):
    mu = cs_ref[0:1, :] * (1.0 / N)
    var = sq_ref[0, 0] * (1.0 / N) - jnp.sum(mu * mu)
    s = jax.lax.rsqrt(EPS + var)
    o_ref[...] = (y_ref[...] - mu) * s


def _scale(y, cs, sq):
    nb = N // ROWBLK
    return pl.pallas_call(
        _scale_body,
        grid=(nb,),
        in_specs=[
            pl.BlockSpec((ROWBLK, HID), lambda g: (g, 0)),
            pl.BlockSpec((8, HID), lambda g: (0, 0)),
            pl.BlockSpec((8, HID), lambda g: (0, 0)),
        ],
        out_specs=pl.BlockSpec((ROWBLK, HID), lambda g: (g, 0)),
        out_shape=jax.ShapeDtypeStruct((N, HID), jnp.float32),
    )(y, cs, sq)


_MESH = plsc.VectorSubcoreMesh(core_axis_name="c", subcore_axis_name="s",
                               num_cores=NCORE, num_subcores=NSUB)
_SLICE = NRP // NSUB


_ZROWS = 157


_NRING = 4


def _edge_kernel(with_deg, t0, t1, t2, t3, gid_hbm, tid_hbm, pp_hbm, deg_hbm,
                 gidv, tidv, rows, zbuf, acc, gsem, ssem,
                 zbuf1=None, ones_v=None, deg_sh=None, dsem=None):
    ci = lax.axis_index("c")
    s = lax.axis_index("s")
    wid = s * NCORE + ci
    tabs = (t0, t1, t2, t3)

    pltpu.sync_copy(gid_hbm.at[wid], gidv)
    pltpu.sync_copy(tid_hbm.at[wid], tidv)

    @pl.loop(0, _ZROWS)
    def _(i):
        zbuf[i, pl.ds(0, 16)] = jnp.zeros((16,), jnp.float32)
        zbuf[i, pl.ds(16, 16)] = jnp.zeros((16,), jnp.float32)
    if with_deg:
        @pl.loop(0, CHUNK // 16)
        def _(i):
            ones_v[pl.ds(i * 16, 16)] = jnp.ones((16,), jnp.float32)
        @pl.loop(0, _SLICE // 16)
        def _(i):
            zbuf1[pl.ds(i * 16, 16)] = jnp.zeros((16,), jnp.float32)
        pltpu.sync_copy(zbuf1, deg_sh.at[pl.ds(s * _SLICE, _SLICE)])
    del wid

    @pl.loop(0, _SLICE // _ZROWS)
    def _(j):
        pltpu.sync_copy(
            zbuf, acc.at[pl.ds(s * _SLICE + j * _ZROWS, _ZROWS)])

    for j in range(_NRING // 2):
        pltpu.async_copy(tabs[0].at[gidv.at[j]], rows[j], gsem[j])

    for p in range(PASSES):
        tab = tabs[p]
        half = _NRING // 2
        plsc.subcore_barrier()

        @pl.loop(0, NCHUNK, step=_NRING)
        def _(k):
            for j in range(_NRING):
                c = k + j
                jj = (j + half) % _NRING
                @pl.when(c + half < NCHUNK)
                def _():
                    @pl.when(c >= half)
                    def _():
                        pltpu.make_async_copy(
                            rows[jj], acc.at[tidv.at[c - half]],
                            ssem[jj]).wait()
                    pltpu.async_copy(tab.at[gidv.at[c + half]],
                                     rows[jj], gsem[jj])

                pltpu.make_async_copy(tab.at[gidv.at[c]], rows[j],
                                      gsem[j]).wait()
                pltpu.async_copy(rows[j], acc.at[tidv.at[c]], ssem[j],
                                 add=True)
                if with_deg and p == 0:
                    @pl.when(c >= 8)
                    def _():
                        pltpu.make_async_copy(ones_v, deg_sh.at[tidv.at[c - 8]],
                                              dsem).wait()
                    pltpu.async_copy(ones_v, deg_sh.at[tidv.at[c]], dsem,
                                     add=True)

        for j in range(_NRING):
            c = NCHUNK - _NRING + j
            pltpu.make_async_copy(rows[c % _NRING], acc.at[tidv.at[c]],
                                  ssem[c % _NRING]).wait()
        if with_deg and p == 0:
            @pl.loop(NCHUNK - 8, NCHUNK)
            def _(k):
                pltpu.make_async_copy(ones_v, deg_sh.at[tidv.at[k]],
                                      dsem).wait()

        if p + 1 < PASSES:
            for j in range(half):
                pltpu.async_copy(tabs[p + 1].at[gidv.at[j]], rows[j],
                                 gsem[j])

        plsc.subcore_barrier()
        pltpu.sync_copy(acc.at[pl.ds(s * _SLICE, _SLICE)],
                        pp_hbm.at[ci, p, pl.ds(s * _SLICE, _SLICE)])
        if with_deg and p == 0:
            pltpu.sync_copy(deg_sh.at[pl.ds(s * _SLICE, _SLICE)],
                            deg_hbm.at[ci, pl.ds(s * _SLICE, _SLICE)])
        if p < PASSES - 1:
            @pl.loop(0, _SLICE // _ZROWS)
            def _(j):
                pltpu.sync_copy(
                    zbuf, acc.at[pl.ds(s * _SLICE + j * _ZROWS, _ZROWS)])


def _edge(tabs, gid3, tid3, with_deg):
    out_type = [jax.ShapeDtypeStruct((NCORE, PASSES, NRP, PCOL), jnp.float32),
                jax.ShapeDtypeStruct((NCORE, NRP), jnp.float32)]
    return pl.kernel(
        functools.partial(_edge_kernel, with_deg),
        out_type=out_type,
        mesh=_MESH,
        compiler_params=pltpu.CompilerParams(use_tc_tiling_on_sc=False),
        scratch_types=[
            pltpu.VMEM((NCHUNK, CHUNK), jnp.int32),
            pltpu.VMEM((NCHUNK, CHUNK), jnp.int32),
            [pltpu.VMEM((CHUNK, PCOL), jnp.float32)] * _NRING,
            pltpu.VMEM((_ZROWS, PCOL), jnp.float32),
            pltpu.VMEM_SHARED((NRP, PCOL), jnp.float32),
            [pltpu.SemaphoreType.DMA] * _NRING,
            [pltpu.SemaphoreType.DMA] * _NRING,
        ] + ([
            pltpu.VMEM((_SLICE,), jnp.float32),
            pltpu.VMEM((CHUNK,), jnp.float32),
            pltpu.VMEM_SHARED((NRP,), jnp.float32),
            pltpu.SemaphoreType.DMA,
        ] if with_deg else []),
    )(tabs[0], tabs[1], tabs[2], tabs[3], gid3, tid3)


def kernel(x_paper, x_author, edge_index, edge_type,
           proj_w_paper, proj_b_paper, proj_w_author, proj_b_author,
           comp1, basis1, root1, comp2, basis2, root2):
    src = edge_index[0]
    dst = edge_index[1]

    gid, tid = _prep_ids(src, dst, edge_type)
    k = jnp.arange(EXTRA, dtype=jnp.int32)
    pad_gid = (k * 7919) % NR
    pad_tid = NR + k % (NRP - NR)
    gid3 = jnp.concatenate([gid.reshape(E), pad_gid]).reshape(
        NWORK, NCHUNK, CHUNK)
    tid3 = jnp.concatenate([tid.reshape(E), pad_tid]).reshape(
        NWORK, NCHUNK, CHUNK)

    xs = jnp.stack([x_paper, x_author])
    ws = jnp.stack([proj_w_paper, proj_w_author])
    bs = jnp.broadcast_to(jnp.stack([proj_b_paper, proj_b_author])[:, None, :],
                          (2, 8, HID))
    x = _proj(xs, ws, bs).reshape(N, HID)

    comps = jnp.stack([comp1, comp2])
    basisr = jnp.stack([basis1.reshape(NB, HID * HID),
                        basis2.reshape(NB, HID * HID)])
    wall = _wmix(comps, basisr)

    def wb(l):
        w = wall[l].reshape(R, HID, PASSES, PCOL).transpose(1, 2, 0, 3)
        return w.reshape(HID, R * HID)

    ys = _hmm(x, wb(0))
    tabs = [y.reshape(NR, PCOL) for y in ys]
    pp, degp = _edge(tabs, gid3, tid3, with_deg=True)
    degx = degp.reshape(NCORE, NPAD, R)
    y1_, cs1, sq1 = _combine(pp.reshape(NCORE, PASSES, NPAD, HID),
                             x, root1, degx, relu=True)

    h, t0, t1, t2, t3 = _hmm_scale(y1_, cs1, sq1, wb(1))
    tabs2 = [t.reshape(NR, PCOL) for t in (t0, t1, t2, t3)]
    pp2, _ = _edge(tabs2, gid3, tid3, with_deg=False)
    y2_, cs2, sq2 = _combine(pp2.reshape(NCORE, PASSES, NPAD, HID),
                             h, root2, degx, relu=False)
    out = _scale(y2_, cs2, sq2)
    return (out, h)


# speedup vs baseline: 21.4187x; 1.0390x over previous
import functools

import jax
import jax.numpy as jnp
from jax import lax
from jax.experimental import pallas as pl
from jax.experimental.pallas import tpu as pltpu
from jax.experimental.pallas import tpu_sc as plsc

N = 10000
E = 320000
R = 4
NB = 8
HID = 128
EPS = 1e-5

NR = N * R
NRP = 40192
NPAD = NRP // R
NCORE = 2
NSUB = 16
NWORK = NCORE * NSUB
EPADDED = 327680
EXTRA = EPADDED - E
CHUNK = 128
NCHUNK = EPADDED // (NWORK * CHUNK)
PASSES = 4
PCOL = HID // PASSES
ROWBLK = 2000


def _prep_body(src_ref, dst_ref, et_ref, gid_ref, tid_ref):
    et = et_ref[...]
    gid_ref[...] = src_ref[...] * R + et
    tid_ref[...] = dst_ref[...] * R + et


def _prep_ids(src, dst, et):
    shp = jax.ShapeDtypeStruct((2500, HID), jnp.int32)
    return pl.pallas_call(
        _prep_body,
        out_shape=(shp, shp),
    )(src.reshape(2500, HID), dst.reshape(2500, HID), et.reshape(2500, HID))


def _wmix_body(comp_ref, basis_ref, w_ref):
    w_ref[0] = jnp.dot(comp_ref[0], basis_ref[0],
                       preferred_element_type=jnp.float32)


def _wmix(comp, basisr):
    return pl.pallas_call(
        _wmix_body,
        grid=(2,),
        in_specs=[
            pl.BlockSpec((1, R, NB), lambda g: (g, 0, 0)),
            pl.BlockSpec((1, NB, HID * HID), lambda g: (g, 0, 0)),
        ],
        out_specs=pl.BlockSpec((1, R, HID * HID), lambda g: (g, 0, 0)),
        out_shape=jax.ShapeDtypeStruct((2, R, HID * HID), jnp.float32),
    )(comp, basisr)


def _proj_body(x_ref, w_ref, b_ref, y_ref):
    y = jnp.dot(x_ref[0], w_ref[0], preferred_element_type=jnp.float32)
    y_ref[0] = jnp.maximum(y + b_ref[0, 0:1, :], 0.0)


def _proj(xs, ws, bs):
    nb = 5
    blk = 5000 // nb
    return pl.pallas_call(
        _proj_body,
        grid=(2, nb),
        in_specs=[
            pl.BlockSpec((1, blk, HID), lambda t, g: (t, g, 0)),
            pl.BlockSpec((1, HID, HID), lambda t, g: (t, 0, 0)),
            pl.BlockSpec((1, 8, HID), lambda t, g: (t, 0, 0)),
        ],
        out_specs=pl.BlockSpec((1, blk, HID), lambda t, g: (t, g, 0)),
        out_shape=jax.ShapeDtypeStruct((2, 5000, HID), jnp.float32),
    )(xs, ws, bs)


def _hmm_body(x_ref, w_ref, y0_ref, y1_ref, y2_ref, y3_ref):
    h = jnp.dot(x_ref[...], w_ref[...], preferred_element_type=jnp.float32)
    y0_ref[...] = h[:, 0:HID]
    y1_ref[...] = h[:, HID:2 * HID]
    y2_ref[...] = h[:, 2 * HID:3 * HID]
    y3_ref[...] = h[:, 3 * HID:4 * HID]


def _hmm(x, wb):
    nb = N // ROWBLK
    shp = jax.ShapeDtypeStruct((N, HID), jnp.float32)
    return pl.pallas_call(
        _hmm_body,
        grid=(nb,),
        in_specs=[
            pl.BlockSpec((ROWBLK, HID), lambda g: (g, 0)),
            pl.BlockSpec((HID, 4 * HID), lambda g: (0, 0)),
        ],
        out_specs=[pl.BlockSpec((ROWBLK, HID), lambda g: (g, 0))] * 4,
        out_shape=(shp,) * 4,
    )(x, wb)


def _hmm_scale_body(y_ref, cs_ref, sq_ref, w_ref, h_ref, y0_ref, y1_ref,
                    y2_ref, y3_ref):
    mu = cs_ref[0:1, :] * (1.0 / N)
    var = sq_ref[0, 0] * (1.0 / N) - jnp.sum(mu * mu)
    sc = jax.lax.rsqrt(EPS + var)
    xb = (y_ref[...] - mu) * sc
    h_ref[...] = xb
    h = jnp.dot(xb, w_ref[...], preferred_element_type=jnp.float32)
    y0_ref[...] = h[:, 0:HID]
    y1_ref[...] = h[:, HID:2 * HID]
    y2_ref[...] = h[:, 2 * HID:3 * HID]
    y3_ref[...] = h[:, 3 * HID:4 * HID]


def _hmm_scale(y, cs, sq, wb):
    nb = N // ROWBLK
    shp = jax.ShapeDtypeStruct((N, HID), jnp.float32)
    return pl.pallas_call(
        _hmm_scale_body,
        grid=(nb,),
        in_specs=[
            pl.BlockSpec((ROWBLK, HID), lambda g: (g, 0)),
            pl.BlockSpec((8, HID), lambda g: (0, 0)),
            pl.BlockSpec((8, HID), lambda g: (0, 0)),
            pl.BlockSpec((HID, 4 * HID), lambda g: (0, 0)),
        ],
        out_specs=[pl.BlockSpec((ROWBLK, HID), lambda g: (g, 0))] * 5,
        out_shape=(shp,) * 5,
    )(y, cs, sq, wb)


def _combine_body(relu, pp_ref, x_ref, root_ref, deg_ref, y_ref, cs_ref,
                  sq_ref):
    g = pl.program_id(0)
    o0 = jnp.dot(x_ref[...], root_ref[...],
                 preferred_element_type=jnp.float32)
    d4 = 1.0 / jnp.maximum(deg_ref[0] + deg_ref[1], 1.0)
    rows = jax.lax.broadcasted_iota(jnp.int32, (R, HID), 0)
    cols = jax.lax.broadcasted_iota(jnp.int32, (R, HID), 1)
    sel = (cols // PCOL == rows).astype(jnp.float32)
    inv = jnp.dot(d4, sel, preferred_element_type=jnp.float32)
    parts = []
    for p in range(PASSES):
        w = (pp_ref[0, p] + pp_ref[1, p]) * inv
        t = w[:, 0:PCOL]
        for r in range(1, R):
            t = t + w[:, r * PCOL:(r + 1) * PCOL]
        parts.append(t)
    y = o0 + jnp.concatenate(parts, axis=1)
    if relu:
        y = jnp.maximum(y, 0.0)
    y_ref[...] = y
    cs = jnp.broadcast_to(jnp.sum(y, axis=0, keepdims=True), (8, HID))
    sq = jnp.full((8, HID), jnp.sum(y * y))

    @pl.when(g == 0)
    def _():
        cs_ref[...] = cs
        sq_ref[...] = sq

    @pl.when(g > 0)
    def _():
        cs_ref[...] += cs
        sq_ref[...] += sq


def _combine(pp, x, root, degx, relu):
    nb = N // ROWBLK
    small = jax.ShapeDtypeStruct((8, HID), jnp.float32)
    return pl.pallas_call(
        functools.partial(_combine_body, relu),
        grid=(nb,),
        in_specs=[
            pl.BlockSpec((2, PASSES, ROWBLK, HID), lambda g: (0, 0, g, 0)),
            pl.BlockSpec((ROWBLK, HID), lambda g: (g, 0)),
            pl.BlockSpec((HID, HID), lambda g: (0, 0)),
            pl.BlockSpec((2, ROWBLK, R), lambda g: (0, g, 0)),
        ],
        out_specs=[
            pl.BlockSpec((ROWBLK, HID), lambda g: (g, 0)),
            pl.BlockSpec((8, HID), lambda g: (0, 0)),
            pl.BlockSpec((8, HID), lambda g: (0, 0)),
        ],
        out_shape=(jax.ShapeDtypeStruct((N, HID), jnp.float32), small, small),
    )(pp, x, root, degx)


def _scale_body(y_ref, cs_ref, sq_ref, o_ref):
    mu = cs_ref[0:1, :] * (1.0 / N)
    var = sq_ref[0, 0] * (1.0 / N) - jnp.sum(mu * mu)
    s = jax.lax.rsqrt(EPS + var)
    o_ref[...] = (y_ref[...] - mu) * s


def _scale(y, cs, sq):
    nb = N // ROWBLK
    return pl.pallas_call(
        _scale_body,
        grid=(nb,),
        in_specs=[
            pl.BlockSpec((ROWBLK, HID), lambda g: (g, 0)),
            pl.BlockSpec((8, HID), lambda g: (0, 0)),
            pl.BlockSpec((8, HID), lambda g: (0, 0)),
        ],
        out_specs=pl.BlockSpec((ROWBLK, HID), lambda g: (g, 0)),
        out_shape=jax.ShapeDtypeStruct((N, HID), jnp.float32),
    )(y, cs, sq)


_MESH = plsc.VectorSubcoreMesh(core_axis_name="c", subcore_axis_name="s",
                               num_cores=NCORE, num_subcores=NSUB)
_SLICE = NRP // NSUB


_ZROWS = 157


_NRING = 4
_PREF = 3


def _edge_kernel(with_deg, t0, t1, t2, t3, gid_hbm, tid_hbm, pp_hbm, deg_hbm,
                 gidv, tidv, rows, zbuf, acc, gsem, ssem,
                 zbuf1=None, ones_v=None, deg_sh=None, dsem=None):
    ci = lax.axis_index("c")
    s = lax.axis_index("s")
    wid = s * NCORE + ci
    tabs = (t0, t1, t2, t3)

    pltpu.sync_copy(gid_hbm.at[wid], gidv)
    pltpu.sync_copy(tid_hbm.at[wid], tidv)

    @pl.loop(0, _ZROWS)
    def _(i):
        zbuf[i, pl.ds(0, 16)] = jnp.zeros((16,), jnp.float32)
        zbuf[i, pl.ds(16, 16)] = jnp.zeros((16,), jnp.float32)
    if with_deg:
        @pl.loop(0, CHUNK // 16)
        def _(i):
            ones_v[pl.ds(i * 16, 16)] = jnp.ones((16,), jnp.float32)
        @pl.loop(0, _SLICE // 16)
        def _(i):
            zbuf1[pl.ds(i * 16, 16)] = jnp.zeros((16,), jnp.float32)
        pltpu.sync_copy(zbuf1, deg_sh.at[pl.ds(s * _SLICE, _SLICE)])
    del wid

    @pl.loop(0, _SLICE // _ZROWS)
    def _(j):
        pltpu.sync_copy(
            zbuf, acc.at[pl.ds(s * _SLICE + j * _ZROWS, _ZROWS)])

    for j in range(_PREF):
        pltpu.async_copy(tabs[0].at[gidv.at[j]], rows[j], gsem[j])

    for p in range(PASSES):
        tab = tabs[p]
        plsc.subcore_barrier()

        @pl.loop(0, NCHUNK, step=_NRING)
        def _(k):
            for j in range(_NRING):
                c = k + j
                jj = (j + _PREF) % _NRING
                @pl.when(c + _PREF < NCHUNK)
                def _():
                    @pl.when(c >= _NRING - _PREF)
                    def _():
                        pltpu.make_async_copy(
                            rows[jj], acc.at[tidv.at[c - (_NRING - _PREF)]],
                            ssem[jj]).wait()
                    pltpu.async_copy(tab.at[gidv.at[c + _PREF]],
                                     rows[jj], gsem[jj])

                pltpu.make_async_copy(tab.at[gidv.at[c]], rows[j],
                                      gsem[j]).wait()
                pltpu.async_copy(rows[j], acc.at[tidv.at[c]], ssem[j],
                                 add=True)
                if with_deg and p == 0:
                    @pl.when(c >= 8)
                    def _():
                        pltpu.make_async_copy(ones_v, deg_sh.at[tidv.at[c - 8]],
                                              dsem).wait()
                    pltpu.async_copy(ones_v, deg_sh.at[tidv.at[c]], dsem,
                                     add=True)

        for j in range(_NRING):
            c = NCHUNK - _NRING + j
            pltpu.make_async_copy(rows[c % _NRING], acc.at[tidv.at[c]],
                                  ssem[c % _NRING]).wait()
        if with_deg and p == 0:
            @pl.loop(NCHUNK - 8, NCHUNK)
            def _(k):
                pltpu.make_async_copy(ones_v, deg_sh.at[tidv.at[k]],
                                      dsem).wait()

        if p + 1 < PASSES:
            for j in range(_PREF):
                pltpu.async_copy(tabs[p + 1].at[gidv.at[j]], rows[j],
                                 gsem[j])

        plsc.subcore_barrier()
        pltpu.sync_copy(acc.at[pl.ds(s * _SLICE, _SLICE)],
                        pp_hbm.at[ci, p, pl.ds(s * _SLICE, _SLICE)])
        if with_deg and p == 0:
            pltpu.sync_copy(deg_sh.at[pl.ds(s * _SLICE, _SLICE)],
                            deg_hbm.at[ci, pl.ds(s * _SLICE, _SLICE)])
        if p < PASSES - 1:
            @pl.loop(0, _SLICE // _ZROWS)
            def _(j):
                pltpu.sync_copy(
                    zbuf, acc.at[pl.ds(s * _SLICE + j * _ZROWS, _ZROWS)])


def _edge(tabs, gid3, tid3, with_deg):
    out_type = [jax.ShapeDtypeStruct((NCORE, PASSES, NRP, PCOL), jnp.float32),
                jax.ShapeDtypeStruct((NCORE, NRP), jnp.float32)]
    return pl.kernel(
        functools.partial(_edge_kernel, with_deg),
        out_type=out_type,
        mesh=_MESH,
        compiler_params=pltpu.CompilerParams(use_tc_tiling_on_sc=False),
        scratch_types=[
            pltpu.VMEM((NCHUNK, CHUNK), jnp.int32),
            pltpu.VMEM((NCHUNK, CHUNK), jnp.int32),
            [pltpu.VMEM((CHUNK, PCOL), jnp.float32)] * _NRING,
            pltpu.VMEM((_ZROWS, PCOL), jnp.float32),
            pltpu.VMEM_SHARED((NRP, PCOL), jnp.float32),
            [pltpu.SemaphoreType.DMA] * _NRING,
            [pltpu.SemaphoreType.DMA] * _NRING,
        ] + ([
            pltpu.VMEM((_SLICE,), jnp.float32),
            pltpu.VMEM((CHUNK,), jnp.float32),
            pltpu.VMEM_SHARED((NRP,), jnp.float32),
            pltpu.SemaphoreType.DMA,
        ] if with_deg else []),
    )(tabs[0], tabs[1], tabs[2], tabs[3], gid3, tid3)


def kernel(x_paper, x_author, edge_index, edge_type,
           proj_w_paper, proj_b_paper, proj_w_author, proj_b_author,
           comp1, basis1, root1, comp2, basis2, root2):
    src = edge_index[0]
    dst = edge_index[1]

    gid, tid = _prep_ids(src, dst, edge_type)
    k = jnp.arange(EXTRA, dtype=jnp.int32)
    pad_gid = (k * 7919) % NR
    pad_tid = NR + k % (NRP - NR)
    gid3 = jnp.concatenate([gid.reshape(E), pad_gid]).reshape(
        NWORK, NCHUNK, CHUNK)
    tid3 = jnp.concatenate([tid.reshape(E), pad_tid]).reshape(
        NWORK, NCHUNK, CHUNK)

    xs = jnp.stack([x_paper, x_author])
    ws = jnp.stack([proj_w_paper, proj_w_author])
    bs = jnp.broadcast_to(jnp.stack([proj_b_paper, proj_b_author])[:, None, :],
                          (2, 8, HID))
    x = _proj(xs, ws, bs).reshape(N, HID)

    comps = jnp.stack([comp1, comp2])
    basisr = jnp.stack([basis1.reshape(NB, HID * HID),
                        basis2.reshape(NB, HID * HID)])
    wall = _wmix(comps, basisr)

    def wb(l):
        w = wall[l].reshape(R, HID, PASSES, PCOL).transpose(1, 2, 0, 3)
        return w.reshape(HID, R * HID)

    ys = _hmm(x, wb(0))
    tabs = [y.reshape(NR, PCOL) for y in ys]
    pp, degp = _edge(tabs, gid3, tid3, with_deg=True)
    degx = degp.reshape(NCORE, NPAD, R)
    y1_, cs1, sq1 = _combine(pp.reshape(NCORE, PASSES, NPAD, HID),
                             x, root1, degx, relu=True)

    h, t0, t1, t2, t3 = _hmm_scale(y1_, cs1, sq1, wb(1))
    tabs2 = [t.reshape(NR, PCOL) for t in (t0, t1, t2, t3)]
    pp2, _ = _edge(tabs2, gid3, tid3, with_deg=False)
    y2_, cs2, sq2 = _combine(pp2.reshape(NCORE, PASSES, NPAD, HID),
                             h, root2, degx, relu=False)
    out = _scale(y2_, cs2, sq2)
    return (out, h)


# speedup vs baseline: 21.9095x; 1.0229x over previous
import functools

import jax
import jax.numpy as jnp
from jax import lax
from jax.experimental import pallas as pl
from jax.experimental.pallas import tpu as pltpu
from jax.experimental.pallas import tpu_sc as plsc

N = 10000
E = 320000
R = 4
NB = 8
HID = 128
EPS = 1e-5

NR = N * R
NRP = 40192
NPAD = NRP // R
NCORE = 2
NSUB = 16
NWORK = NCORE * NSUB
EPADDED = 327680
EXTRA = EPADDED - E
CHUNK = 128
NCHUNK = EPADDED // (NWORK * CHUNK)
PASSES = 4
PCOL = HID // PASSES
ROWBLK = 2000


def _prep_body(src_ref, dst_ref, et_ref, gid_ref, tid_ref):
    et = et_ref[...]
    gid_ref[...] = src_ref[...] * R + et
    tid_ref[...] = dst_ref[...] * R + et


def _prep_ids(src, dst, et):
    shp = jax.ShapeDtypeStruct((2500, HID), jnp.int32)
    return pl.pallas_call(
        _prep_body,
        out_shape=(shp, shp),
    )(src.reshape(2500, HID), dst.reshape(2500, HID), et.reshape(2500, HID))


def _wmix_body(comp_ref, basis_ref, w_ref):
    w_ref[0] = jnp.dot(comp_ref[0], basis_ref[0],
                       preferred_element_type=jnp.float32)


def _wmix(comp, basisr):
    return pl.pallas_call(
        _wmix_body,
        grid=(2,),
        in_specs=[
            pl.BlockSpec((1, R, NB), lambda g: (g, 0, 0)),
            pl.BlockSpec((1, NB, HID * HID), lambda g: (g, 0, 0)),
        ],
        out_specs=pl.BlockSpec((1, R, HID * HID), lambda g: (g, 0, 0)),
        out_shape=jax.ShapeDtypeStruct((2, R, HID * HID), jnp.float32),
    )(comp, basisr)


def _proj_body(x_ref, w_ref, b_ref, y_ref):
    y = jnp.dot(x_ref[0], w_ref[0], preferred_element_type=jnp.float32)
    y_ref[0] = jnp.maximum(y + b_ref[0, 0:1, :], 0.0)


def _proj(xs, ws, bs):
    nb = 5
    blk = 5000 // nb
    return pl.pallas_call(
        _proj_body,
        grid=(2, nb),
        in_specs=[
            pl.BlockSpec((1, blk, HID), lambda t, g: (t, g, 0)),
            pl.BlockSpec((1, HID, HID), lambda t, g: (t, 0, 0)),
            pl.BlockSpec((1, 8, HID), lambda t, g: (t, 0, 0)),
        ],
        out_specs=pl.BlockSpec((1, blk, HID), lambda t, g: (t, g, 0)),
        out_shape=jax.ShapeDtypeStruct((2, 5000, HID), jnp.float32),
    )(xs, ws, bs)


def _hmm_body(x_ref, w_ref, y0_ref, y1_ref, y2_ref, y3_ref):
    h = jnp.dot(x_ref[...], w_ref[...], preferred_element_type=jnp.float32)
    y0_ref[...] = h[:, 0:HID]
    y1_ref[...] = h[:, HID:2 * HID]
    y2_ref[...] = h[:, 2 * HID:3 * HID]
    y3_ref[...] = h[:, 3 * HID:4 * HID]


def _hmm(x, wb):
    nb = N // ROWBLK
    shp = jax.ShapeDtypeStruct((N, HID), jnp.float32)
    return pl.pallas_call(
        _hmm_body,
        grid=(nb,),
        in_specs=[
            pl.BlockSpec((ROWBLK, HID), lambda g: (g, 0)),
            pl.BlockSpec((HID, 4 * HID), lambda g: (0, 0)),
        ],
        out_specs=[pl.BlockSpec((ROWBLK, HID), lambda g: (g, 0))] * 4,
        out_shape=(shp,) * 4,
    )(x, wb)


def _hmm_scale_body(y_ref, cs_ref, sq_ref, w_ref, h_ref, y0_ref, y1_ref,
                    y2_ref, y3_ref):
    mu = cs_ref[0:1, :] * (1.0 / N)
    var = sq_ref[0, 0] * (1.0 / N) - jnp.sum(mu * mu)
    sc = jax.lax.rsqrt(EPS + var)
    xb = (y_ref[...] - mu) * sc
    h_ref[...] = xb
    h = jnp.dot(xb, w_ref[...], preferred_element_type=jnp.float32)
    y0_ref[...] = h[:, 0:HID]
    y1_ref[...] = h[:, HID:2 * HID]
    y2_ref[...] = h[:, 2 * HID:3 * HID]
    y3_ref[...] = h[:, 3 * HID:4 * HID]


def _hmm_scale(y, cs, sq, wb):
    nb = N // ROWBLK
    shp = jax.ShapeDtypeStruct((N, HID), jnp.float32)
    return pl.pallas_call(
        _hmm_scale_body,
        grid=(nb,),
        in_specs=[
            pl.BlockSpec((ROWBLK, HID), lambda g: (g, 0)),
            pl.BlockSpec((8, HID), lambda g: (0, 0)),
            pl.BlockSpec((8, HID), lambda g: (0, 0)),
            pl.BlockSpec((HID, 4 * HID), lambda g: (0, 0)),
        ],
        out_specs=[pl.BlockSpec((ROWBLK, HID), lambda g: (g, 0))] * 5,
        out_shape=(shp,) * 5,
    )(y, cs, sq, wb)


def _combine_body(relu, pp_ref, x_ref, root_ref, deg_ref, y_ref, cs_ref,
                  sq_ref):
    g = pl.program_id(0)
    o0 = jnp.dot(x_ref[...], root_ref[...],
                 preferred_element_type=jnp.float32)
    d4 = 1.0 / jnp.maximum(deg_ref[0] + deg_ref[1], 1.0)
    rows = jax.lax.broadcasted_iota(jnp.int32, (R, HID), 0)
    cols = jax.lax.broadcasted_iota(jnp.int32, (R, HID), 1)
    sel = (cols // PCOL == rows).astype(jnp.float32)
    inv = jnp.dot(d4, sel, preferred_element_type=jnp.float32)
    parts = []
    for p in range(PASSES):
        w = (pp_ref[0, p] + pp_ref[1, p]) * inv
        t = w[:, 0:PCOL]
        for r in range(1, R):
            t = t + w[:, r * PCOL:(r + 1) * PCOL]
        parts.append(t)
    y = o0 + jnp.concatenate(parts, axis=1)
    if relu:
        y = jnp.maximum(y, 0.0)
    y_ref[...] = y
    cs = jnp.broadcast_to(jnp.sum(y, axis=0, keepdims=True), (8, HID))
    sq = jnp.full((8, HID), jnp.sum(y * y))

    @pl.when(g == 0)
    def _():
        cs_ref[...] = cs
        sq_ref[...] = sq

    @pl.when(g > 0)
    def _():
        cs_ref[...] += cs
        sq_ref[...] += sq


def _combine(pp, x, root, degx, relu):
    nb = N // ROWBLK
    small = jax.ShapeDtypeStruct((8, HID), jnp.float32)
    return pl.pallas_call(
        functools.partial(_combine_body, relu),
        grid=(nb,),
        in_specs=[
            pl.BlockSpec((2, PASSES, ROWBLK, HID), lambda g: (0, 0, g, 0)),
            pl.BlockSpec((ROWBLK, HID), lambda g: (g, 0)),
            pl.BlockSpec((HID, HID), lambda g: (0, 0)),
            pl.BlockSpec((2, ROWBLK, R), lambda g: (0, g, 0)),
        ],
        out_specs=[
            pl.BlockSpec((ROWBLK, HID), lambda g: (g, 0)),
            pl.BlockSpec((8, HID), lambda g: (0, 0)),
            pl.BlockSpec((8, HID), lambda g: (0, 0)),
        ],
        out_shape=(jax.ShapeDtypeStruct((N, HID), jnp.float32), small, small),
    )(pp, x, root, degx)


def _scale_body(y_ref, cs_ref, sq_ref, o_ref):
    mu = cs_ref[0:1, :] * (1.0 / N)
    var = sq_ref[0, 0] * (1.0 / N) - jnp.sum(mu * mu)
    s = jax.lax.rsqrt(EPS + var)
    o_ref[...] = (y_ref[...] - mu) * s


def _scale(y, cs, sq):
    nb = N // ROWBLK
    return pl.pallas_call(
        _scale_body,
        grid=(nb,),
        in_specs=[
            pl.BlockSpec((ROWBLK, HID), lambda g: (g, 0)),
            pl.BlockSpec((8, HID), lambda g: (0, 0)),
            pl.BlockSpec((8, HID), lambda g: (0, 0)),
        ],
        out_specs=pl.BlockSpec((ROWBLK, HID), lambda g: (g, 0)),
        out_shape=jax.ShapeDtypeStruct((N, HID), jnp.float32),
    )(y, cs, sq)


_MESH = plsc.VectorSubcoreMesh(core_axis_name="c", subcore_axis_name="s",
                               num_cores=NCORE, num_subcores=NSUB)
_SLICE = NRP // NSUB


_ZROWS = 157


_NRING1 = 4
_NRING2 = 5


def _edge_kernel(with_deg, nring, t0, t1, t2, t3, gid_hbm, tid_hbm, pp_hbm,
                 deg_hbm, gidv, tidv, rows, zbuf, acc, gsem, ssem,
                 zbuf1=None, ones_v=None, deg_sh=None, dsem=None):
    pref = nring - 1
    ci = lax.axis_index("c")
    s = lax.axis_index("s")
    wid = s * NCORE + ci
    tabs = (t0, t1, t2, t3)

    pltpu.sync_copy(gid_hbm.at[wid], gidv)
    pltpu.sync_copy(tid_hbm.at[wid], tidv)

    @pl.loop(0, _ZROWS)
    def _(i):
        zbuf[i, pl.ds(0, 16)] = jnp.zeros((16,), jnp.float32)
        zbuf[i, pl.ds(16, 16)] = jnp.zeros((16,), jnp.float32)
    if with_deg:
        @pl.loop(0, CHUNK // 16)
        def _(i):
            ones_v[pl.ds(i * 16, 16)] = jnp.ones((16,), jnp.float32)
        @pl.loop(0, _SLICE // 16)
        def _(i):
            zbuf1[pl.ds(i * 16, 16)] = jnp.zeros((16,), jnp.float32)
        pltpu.sync_copy(zbuf1, deg_sh.at[pl.ds(s * _SLICE, _SLICE)])
    del wid

    @pl.loop(0, _SLICE // _ZROWS)
    def _(j):
        pltpu.sync_copy(
            zbuf, acc.at[pl.ds(s * _SLICE + j * _ZROWS, _ZROWS)])

    for j in range(pref):
        pltpu.async_copy(tabs[0].at[gidv.at[j]], rows[j], gsem[j])

    for p in range(PASSES):
        tab = tabs[p]
        plsc.subcore_barrier()

        @pl.loop(0, NCHUNK, step=nring)
        def _(k):
            for j in range(nring):
                c = k + j
                jj = (j + pref) % nring
                @pl.when(c + pref < NCHUNK)
                def _():
                    @pl.when(c >= nring - pref)
                    def _():
                        pltpu.make_async_copy(
                            rows[jj], acc.at[tidv.at[c - (nring - pref)]],
                            ssem[jj]).wait()
                    pltpu.async_copy(tab.at[gidv.at[c + pref]],
                                     rows[jj], gsem[jj])

                pltpu.make_async_copy(tab.at[gidv.at[c]], rows[j],
                                      gsem[j]).wait()
                pltpu.async_copy(rows[j], acc.at[tidv.at[c]], ssem[j],
                                 add=True)
                if with_deg and p == 0:
                    @pl.when(c >= 8)
                    def _():
                        pltpu.make_async_copy(ones_v, deg_sh.at[tidv.at[c - 8]],
                                              dsem).wait()
                    pltpu.async_copy(ones_v, deg_sh.at[tidv.at[c]], dsem,
                                     add=True)

        for j in range(nring):
            c = NCHUNK - nring + j
            pltpu.make_async_copy(rows[c % nring], acc.at[tidv.at[c]],
                                  ssem[c % nring]).wait()
        if with_deg and p == 0:
            @pl.loop(NCHUNK - 8, NCHUNK)
            def _(k):
                pltpu.make_async_copy(ones_v, deg_sh.at[tidv.at[k]],
                                      dsem).wait()

        if p + 1 < PASSES:
            for j in range(pref):
                pltpu.async_copy(tabs[p + 1].at[gidv.at[j]], rows[j],
                                 gsem[j])

        plsc.subcore_barrier()
        pltpu.sync_copy(acc.at[pl.ds(s * _SLICE, _SLICE)],
                        pp_hbm.at[ci, p, pl.ds(s * _SLICE, _SLICE)])
        if with_deg and p == 0:
            pltpu.sync_copy(deg_sh.at[pl.ds(s * _SLICE, _SLICE)],
                            deg_hbm.at[ci, pl.ds(s * _SLICE, _SLICE)])
        if p < PASSES - 1:
            @pl.loop(0, _SLICE // _ZROWS)
            def _(j):
                pltpu.sync_copy(
                    zbuf, acc.at[pl.ds(s * _SLICE + j * _ZROWS, _ZROWS)])


def _edge(tabs, gid3, tid3, with_deg):
    nring = _NRING1 if with_deg else _NRING2
    out_type = [jax.ShapeDtypeStruct((NCORE, PASSES, NRP, PCOL), jnp.float32),
                jax.ShapeDtypeStruct((NCORE, NRP), jnp.float32)]
    return pl.kernel(
        functools.partial(_edge_kernel, with_deg, nring),
        out_type=out_type,
        mesh=_MESH,
        compiler_params=pltpu.CompilerParams(use_tc_tiling_on_sc=False),
        scratch_types=[
            pltpu.VMEM((NCHUNK, CHUNK), jnp.int32),
            pltpu.VMEM((NCHUNK, CHUNK), jnp.int32),
            [pltpu.VMEM((CHUNK, PCOL), jnp.float32)] * nring,
            pltpu.VMEM((_ZROWS, PCOL), jnp.float32),
            pltpu.VMEM_SHARED((NRP, PCOL), jnp.float32),
            [pltpu.SemaphoreType.DMA] * nring,
            [pltpu.SemaphoreType.DMA] * nring,
        ] + ([
            pltpu.VMEM((_SLICE,), jnp.float32),
            pltpu.VMEM((CHUNK,), jnp.float32),
            pltpu.VMEM_SHARED((NRP,), jnp.float32),
            pltpu.SemaphoreType.DMA,
        ] if with_deg else []),
    )(tabs[0], tabs[1], tabs[2], tabs[3], gid3, tid3)


def kernel(x_paper, x_author, edge_index, edge_type,
           proj_w_paper, proj_b_paper, proj_w_author, proj_b_author,
           comp1, basis1, root1, comp2, basis2, root2):
    src = edge_index[0]
    dst = edge_index[1]

    gid, tid = _prep_ids(src, dst, edge_type)
    k = jnp.arange(EXTRA, dtype=jnp.int32)
    pad_gid = (k * 7919) % NR
    pad_tid = NR + k % (NRP - NR)
    gid3 = jnp.concatenate([gid.reshape(E), pad_gid]).reshape(
        NWORK, NCHUNK, CHUNK)
    tid3 = jnp.concatenate([tid.reshape(E), pad_tid]).reshape(
        NWORK, NCHUNK, CHUNK)

    xs = jnp.stack([x_paper, x_author])
    ws = jnp.stack([proj_w_paper, proj_w_author])
    bs = jnp.broadcast_to(jnp.stack([proj_b_paper, proj_b_author])[:, None, :],
                          (2, 8, HID))
    x = _proj(xs, ws, bs).reshape(N, HID)

    comps = jnp.stack([comp1, comp2])
    basisr = jnp.stack([basis1.reshape(NB, HID * HID),
                        basis2.reshape(NB, HID * HID)])
    wall = _wmix(comps, basisr)

    def wb(l):
        w = wall[l].reshape(R, HID, PASSES, PCOL).transpose(1, 2, 0, 3)
        return w.reshape(HID, R * HID)

    ys = _hmm(x, wb(0))
    tabs = [y.reshape(NR, PCOL) for y in ys]
    pp, degp = _edge(tabs, gid3, tid3, with_deg=True)
    degx = degp.reshape(NCORE, NPAD, R)
    y1_, cs1, sq1 = _combine(pp.reshape(NCORE, PASSES, NPAD, HID),
                             x, root1, degx, relu=True)

    h, t0, t1, t2, t3 = _hmm_scale(y1_, cs1, sq1, wb(1))
    tabs2 = [t.reshape(NR, PCOL) for t in (t0, t1, t2, t3)]
    pp2, _ = _edge(tabs2, gid3, tid3, with_deg=False)
    y2_, cs2, sq2 = _combine(pp2.reshape(NCORE, PASSES, NPAD, HID),
                             h, root2, degx, relu=False)
    out = _scale(y2_, cs2, sq2)
    return (out, h)
